# Initial kernel scaffold; baseline (speedup 1.0000x reference)
import jax
import jax.numpy as jnp
from jax.experimental import pallas as pl


def kernel(x, edge_index, edge_attr, batch, edge_score, c1_w1, c1_b1, c1_w2, c1_w3, c1_b3, c2_w1, c2_b1, c2_w2, c2_w3, c2_b3):
    raise NotImplementedError("write your pallas kernel here")


# speedup vs baseline: 15.6140x; 15.6140x over previous
import functools
import jax
import jax.numpy as jnp
from jax import lax
from jax.experimental import pallas as pl
from jax.experimental.pallas import tpu as pltpu, tpu_sc as plsc

_N = 10000
_E = 320000
_C = 128
_R = 224000
_KEYMAX = 0x3F7FFFFF

_MESH = plsc.VectorSubcoreMesh(core_axis_name="c", subcore_axis_name="s")
_SCP = pltpu.CompilerParams(needs_layout_passes=False, use_tc_tiling_on_sc=False)

_LANES = lambda: lax.iota(jnp.int32, 16)
_ZI = lambda: jnp.zeros((16,), jnp.int32)
_ZF = lambda: jnp.zeros((16,), jnp.float32)


def _wid():
    return lax.axis_index("c") * 16 + lax.axis_index("s")


def _zero_vmem(ref, n, dtype):
    z = jnp.zeros((16,), dtype)

    def body(i, c):
        ref[pl.ds(i * 16, 16)] = z
        return c

    lax.fori_loop(0, n // 16, body, 0)


_CH1 = 1000


@functools.partial(
    pl.kernel, mesh=_MESH, compiler_params=_SCP,
    out_type=(jax.ShapeDtypeStruct((2, _N, 16), jnp.float32),
              jax.ShapeDtypeStruct((2, _N), jnp.int32)),
    scratch_types=[
        pltpu.VMEM((_CH1,), jnp.int32),
        pltpu.VMEM((_CH1,), jnp.int32),
        pltpu.VMEM((_CH1,), jnp.float32),
        pltpu.VMEM((_CH1, 16), jnp.float32),
        pltpu.VMEM((_CH1,), jnp.int32),
        pltpu.VMEM((640, 16), jnp.float32),
        pltpu.VMEM((640,), jnp.int32),
        pltpu.VMEM_SHARED((_N, 16), jnp.float32),
        pltpu.VMEM_SHARED((_N,), jnp.int32),
        pltpu.SemaphoreType.DMA,
    ],
)
def _k1(xe_hbm, ei_hbm, ew_hbm, acc_hbm, pres_hbm,
        srcv, dstv, ewv, rowsv, onesv, zrows, zi640, acc_sh, pres_sh, sem):
    cid = lax.axis_index("c")
    sid = lax.axis_index("s")
    w = _wid()
    lanes = _LANES()
    def zr(i, c):
        zrows[i, :] = _ZF()
        return c
    lax.fori_loop(0, 640, zr, 0)
    _zero_vmem(zi640, 640, jnp.int32)
    pltpu.sync_copy(zrows.at[pl.ds(0, 625)], acc_sh.at[pl.ds(sid * 625, 625)])

    @pl.when(sid < 15)
    def _():
        pltpu.sync_copy(zi640, pres_sh.at[pl.ds(sid * 640, 640)])

    @pl.when(sid == 15)
    def _():
        pltpu.sync_copy(zi640.at[pl.ds(0, 400)], pres_sh.at[pl.ds(9600, 400)])

    _zero_vmem(onesv, _CH1, jnp.int32)
    def o1(i, c):
        onesv[pl.ds(i * 16, 16)] = _ZI() + 1
        return c
    lax.fori_loop(0, _CH1 // 16, o1, 0)
    plsc.subcore_barrier()

    base_w = w * (_E // 32)
    for j in range((_E // 32) // _CH1):
        off = base_w + j * _CH1
        pltpu.sync_copy(ei_hbm.at[0, pl.ds(off, _CH1)], srcv)
        pltpu.sync_copy(ei_hbm.at[1, pl.ds(off, _CH1)], dstv)
        pltpu.sync_copy(ew_hbm.at[pl.ds(off, _CH1)], ewv)
        pltpu.async_copy(xe_hbm.at[srcv], rowsv, sem).wait()

        def mul(r, c):
            s = plsc.load_gather(ewv, [_ZI() + r])
            rowsv[r, :] = rowsv[r, :] * s
            return c

        lax.fori_loop(0, _CH1, mul, 0)
        pltpu.sync_copy(rowsv, acc_sh.at[dstv], add=True)
        pltpu.sync_copy(onesv, pres_sh.at[srcv], add=True)
        pltpu.sync_copy(onesv, pres_sh.at[dstv], add=True)

    plsc.subcore_barrier()
    pltpu.sync_copy(acc_sh.at[pl.ds(sid * 625, 625)],
                    acc_hbm.at[cid, pl.ds(sid * 625, 625)])

    @pl.when(sid < 15)
    def _():
        pltpu.sync_copy(pres_sh.at[pl.ds(sid * 640, 640)],
                        pres_hbm.at[cid, pl.ds(sid * 640, 640)])

    @pl.when(sid == 15)
    def _():
        pltpu.sync_copy(pres_sh.at[pl.ds(9600, 400)],
                        pres_hbm.at[cid, pl.ds(9600, 400)])


_CH3 = 200


@functools.partial(
    pl.kernel, mesh=_MESH, compiler_params=_SCP,
    out_type=jax.ShapeDtypeStruct((2, _N, _C), jnp.float32),
    scratch_types=[
        pltpu.VMEM((_CH3,), jnp.int32),
        pltpu.VMEM((_CH3,), jnp.int32),
        pltpu.VMEM((_CH3,), jnp.float32),
        pltpu.VMEM((_CH3, _C), jnp.float32),
        pltpu.VMEM_SHARED((_N, _C), jnp.float32),
        pltpu.SemaphoreType.DMA,
    ],
)
def _k3(h1_hbm, ei_hbm, ew_hbm, acc_hbm, srcv, dstv, ewv, rowsv, acc_sh, sem):
    cid = lax.axis_index("c")
    sid = lax.axis_index("s")
    w = _wid()
    def zr(i, c):
        for jj in range(8):
            rowsv[i, pl.ds(jj * 16, 16)] = _ZF()
        return c
    lax.fori_loop(0, _CH3, zr, 0)
    for z in range(3):
        pltpu.sync_copy(rowsv, acc_sh.at[pl.ds(sid * 625 + z * _CH3, _CH3)])
    pltpu.sync_copy(rowsv.at[pl.ds(0, 25)],
                    acc_sh.at[pl.ds(sid * 625 + 3 * _CH3, 25)])
    plsc.subcore_barrier()

    base_w = w * (_E // 32)
    for j in range((_E // 32) // _CH3):
        off = base_w + j * _CH3
        pltpu.sync_copy(ei_hbm.at[0, pl.ds(off, _CH3)], srcv)
        pltpu.sync_copy(ei_hbm.at[1, pl.ds(off, _CH3)], dstv)
        pltpu.sync_copy(ew_hbm.at[pl.ds(off, _CH3)], ewv)
        pltpu.async_copy(h1_hbm.at[srcv], rowsv, sem).wait()

        def mul(r, c):
            s = plsc.load_gather(ewv, [_ZI() + r])
            for jj in range(8):
                rowsv[r, pl.ds(jj * 16, 16)] = rowsv[r, pl.ds(jj * 16, 16)] * s
            return c

        lax.fori_loop(0, _CH3, mul, 0)
        pltpu.sync_copy(rowsv, acc_sh.at[dstv], add=True)

    plsc.subcore_barrier()
    pltpu.sync_copy(acc_sh.at[pl.ds(sid * 625, 625)],
                    acc_hbm.at[cid, pl.ds(sid * 625, 625)])


def _k2_body(acc_ref, x_ref, w1_ref, w2_ref, w3_ref, h1_ref):
    acc = acc_ref[0] + acc_ref[1]
    deg = acc[:, 4:5]
    x = x_ref[...]
    t1 = jnp.dot(acc, w1_ref[...], preferred_element_type=jnp.float32)
    t2 = jnp.dot(x, w2_ref[...], preferred_element_type=jnp.float32)
    t3 = jnp.dot(x, w3_ref[...], preferred_element_type=jnp.float32)
    h1_ref[...] = jnp.maximum(t1 - deg * t2 + t3, 0.0)


_k2 = pl.pallas_call(
    _k2_body,
    grid=(5,),
    in_specs=[
        pl.BlockSpec((2, 2000, 16), lambda i: (0, i, 0)),
        pl.BlockSpec((2000, 16), lambda i: (i, 0)),
        pl.BlockSpec((16, _C), lambda i: (0, 0)),
        pl.BlockSpec((16, _C), lambda i: (0, 0)),
        pl.BlockSpec((16, _C), lambda i: (0, 0)),
    ],
    out_specs=pl.BlockSpec((2000, _C), lambda i: (i, 0)),
    out_shape=jax.ShapeDtypeStruct((_N, _C), jnp.float32),
)


def _k4_body(acc2_ref, acc1_ref, h1_ref, w1_ref, w2_ref, w3_ref, b1_ref,
             b3_ref, h_ref):
    z2 = acc2_ref[0] + acc2_ref[1]
    deg = acc1_ref[0][:, 4:5] + acc1_ref[1][:, 4:5]
    h1 = h1_ref[...]
    t1 = jnp.dot(z2, w1_ref[...], preferred_element_type=jnp.float32)
    t2 = jnp.dot(h1, w2_ref[...], preferred_element_type=jnp.float32)
    t3 = jnp.dot(h1, w3_ref[...], preferred_element_type=jnp.float32)
    h_ref[...] = t1 + deg * b1_ref[...] - deg * t2 + t3 + b3_ref[...]


_k4 = pl.pallas_call(
    _k4_body,
    grid=(5,),
    in_specs=[
        pl.BlockSpec((2, 2000, _C), lambda i: (0, i, 0)),
        pl.BlockSpec((2, 2000, 16), lambda i: (0, i, 0)),
        pl.BlockSpec((2000, _C), lambda i: (i, 0)),
        pl.BlockSpec((_C, _C), lambda i: (0, 0)),
        pl.BlockSpec((_C, _C), lambda i: (0, 0)),
        pl.BlockSpec((_C, _C), lambda i: (0, 0)),
        pl.BlockSpec((1, _C), lambda i: (0, 0)),
        pl.BlockSpec((1, _C), lambda i: (0, 0)),
    ],
    out_specs=pl.BlockSpec((2000, _C), lambda i: (i, 0)),
    out_shape=jax.ShapeDtypeStruct((_N, _C), jnp.float32),
)


_W5 = _E // 16
_S5 = _W5 // 16
_RDX = 256
_KCH = 2000


@functools.partial(
    pl.kernel, mesh=_MESH, compiler_params=_SCP,
    out_type=(jax.ShapeDtypeStruct((_R,), jnp.float32),
              jax.ShapeDtypeStruct((_R,), jnp.float32),
              jax.ShapeDtypeStruct((_R,), jnp.int32),
              jax.ShapeDtypeStruct((_R,), jnp.int32),
              jax.ShapeDtypeStruct((_N,), jnp.int32)),
    scratch_types=[
        pltpu.VMEM((_W5,), jnp.float32),
        pltpu.VMEM((_W5,), jnp.int32),
        pltpu.VMEM((_W5,), jnp.int32),
        pltpu.VMEM((16 * _RDX,), jnp.int32),
        pltpu.VMEM((16 * _RDX,), jnp.int32),
        pltpu.VMEM((16 * _RDX,), jnp.int32),
        pltpu.VMEM((_RDX,), jnp.int32),
        pltpu.VMEM((_KCH,), jnp.float32),
        pltpu.VMEM((_KCH,), jnp.int32),
        pltpu.VMEM((_KCH,), jnp.int32),
        pltpu.VMEM((_KCH,), jnp.int32),
        pltpu.VMEM((640,), jnp.int32),
        pltpu.VMEM_SHARED((_E,), jnp.int32),
        pltpu.VMEM_SHARED((_E,), jnp.int32),
        pltpu.VMEM_SHARED((16 * _RDX,), jnp.int32),
        pltpu.VMEM_SHARED((_N,), jnp.int32),
        pltpu.SemaphoreType.DMA,
    ],
)
def _k5(score_hbm, ew_hbm, src_hbm, dst_hbm,
        rew_hbm, rea_hbm, rsrc_hbm, rdst_hbm, presr_hbm,
        keysf, valsv, posv, histv, offv, Tallv, Tv, s2k, i2k, j2k, ones2k,
        zi640, A_v, B_v, Tall_sh, pres_r_sh, sem):
    cid = lax.axis_index("c")
    sid = lax.axis_index("s")
    lanes = _LANES()

    @pl.when(cid == 0)
    def _():
        base = sid * _W5
        _zero_vmem(zi640, 640, jnp.int32)

        @pl.when(sid < 15)
        def _():
            pltpu.sync_copy(zi640, pres_r_sh.at[pl.ds(sid * 640, 640)])

        @pl.when(sid == 15)
        def _():
            pltpu.sync_copy(zi640.at[pl.ds(0, 400)],
                            pres_r_sh.at[pl.ds(9600, 400)])

        def o1(i, c):
            ones2k[pl.ds(i * 16, 16)] = _ZI() + 1
            return c
        lax.fori_loop(0, _KCH // 16, o1, 0)

        for p in range(4):
            shift = 8 * p
            if p % 2 == 0:
                src_v, dst_v = A_v, B_v
            else:
                src_v, dst_v = B_v, A_v
            if p == 0:
                pltpu.sync_copy(score_hbm.at[pl.ds(base, _W5)], keysf)

                def iv(i, c):
                    valsv[pl.ds(i * 16, 16)] = base + i * 16 + lanes
                    return c
                lax.fori_loop(0, _W5 // 16, iv, 0)
            else:
                pltpu.sync_copy(src_v.at[pl.ds(base, _W5)], valsv)
                pltpu.async_copy(score_hbm.at[valsv], keysf, sem).wait()

            def mk(i, c):
                sl = pl.ds(i * 16, 16)
                keysf[sl] = plsc.bitcast(
                    _KEYMAX - plsc.bitcast(keysf[sl], jnp.int32), jnp.float32)
                return c
            lax.fori_loop(0, _W5 // 16, mk, 0)

            def zh(i, c):
                histv[pl.ds(i * 16, 16)] = _ZI()
                return c
            lax.fori_loop(0, _RDX, zh, 0)

            def ha(t, c):
                kv = plsc.bitcast(plsc.load_gather(keysf, [lanes * _S5 + t]),
                                  jnp.int32)
                d = jnp.bitwise_and(lax.shift_right_logical(kv, shift),
                                    _RDX - 1)
                plsc.addupdate_scatter(histv, [lanes * _RDX + d], _ZI() + 1)
                return c
            lax.fori_loop(0, _S5, ha, 0)

            def wt(j, c):
                acc = _ZI()
                for l in range(16):
                    acc = acc + histv[pl.ds(l * _RDX + j * 16, 16)]
                Tv[pl.ds(j * 16, 16)] = acc
                return c
            lax.fori_loop(0, _RDX // 16, wt, 0)
            pltpu.sync_copy(Tv, Tall_sh.at[pl.ds(sid * _RDX, _RDX)])
            plsc.subcore_barrier()

            pltpu.sync_copy(Tall_sh, Tallv)

            def pb(j, carry):
                G = _ZI()
                Wp = _ZI()
                for wk in range(16):
                    s_w = Tallv[pl.ds(wk * _RDX + j * 16, 16)]
                    G = G + s_w
                    Wp = Wp + s_w * jnp.where(wk < sid, 1, 0)
                cs = plsc.cumsum(G)
                Px = carry + cs - G
                run = Px + Wp
                for l in range(16):
                    offv[pl.ds(l * _RDX + j * 16, 16)] = run
                    run = run + histv[pl.ds(l * _RDX + j * 16, 16)]
                return carry + jnp.sum(G)
            lax.fori_loop(0, _RDX // 16, pb, jnp.int32(0))

            def pc(t, c):
                sl = lanes * _S5 + t
                kv = plsc.bitcast(plsc.load_gather(keysf, [sl]), jnp.int32)
                d = jnp.bitwise_and(lax.shift_right_logical(kv, shift),
                                    _RDX - 1)
                fl = lanes * _RDX + d
                pos = plsc.load_gather(offv, [fl])
                plsc.store_scatter(offv, [fl], pos + 1)
                plsc.store_scatter(posv, [sl], pos)
                return c
            lax.fori_loop(0, _S5, pc, 0)

            pltpu.sync_copy(valsv, dst_v.at[posv])
            plsc.subcore_barrier()

        base2 = sid * (_R // 16)
        for q in range((_R // 16) // _KCH):
            off2 = base2 + q * _KCH
            pltpu.sync_copy(A_v.at[pl.ds(off2, _KCH)], i2k)
            pltpu.async_copy(score_hbm.at[i2k], s2k, sem).wait()
            pltpu.sync_copy(s2k, rew_hbm.at[pl.ds(off2, _KCH)])
            pltpu.async_copy(ew_hbm.at[i2k], s2k, sem).wait()
            pltpu.sync_copy(s2k, rea_hbm.at[pl.ds(off2, _KCH)])
            pltpu.async_copy(src_hbm.at[i2k], j2k, sem).wait()
            pltpu.sync_copy(j2k, rsrc_hbm.at[pl.ds(off2, _KCH)])
            pltpu.sync_copy(ones2k, pres_r_sh.at[j2k], add=True)
            pltpu.async_copy(dst_hbm.at[i2k], j2k, sem).wait()
            pltpu.sync_copy(j2k, rdst_hbm.at[pl.ds(off2, _KCH)])
            pltpu.sync_copy(ones2k, pres_r_sh.at[j2k], add=True)

        plsc.subcore_barrier()

        @pl.when(sid < 15)
        def _():
            pltpu.sync_copy(pres_r_sh.at[pl.ds(sid * 640, 640)],
                            presr_hbm.at[pl.ds(sid * 640, 640)])

        @pl.when(sid == 15)
        def _():
            pltpu.sync_copy(pres_r_sh.at[pl.ds(9600, 400)],
                            presr_hbm.at[pl.ds(9600, 400)])


def _k7_body(pf_ref, pr_ref, nif_ref, nir_ref):
    r128 = lax.broadcasted_iota(jnp.int32, (_C, _C), 0)
    c128 = lax.broadcasted_iota(jnp.int32, (_C, _C), 1)
    Lm = (r128 <= c128).astype(jnp.float32)
    Jm = jnp.ones((_C, _C), jnp.float32)
    r80 = lax.broadcasted_iota(jnp.int32, (80, 80), 0)
    c80 = lax.broadcasted_iota(jnp.int32, (80, 80), 1)
    SL = (c80 < r80).astype(jnp.float32)

    def ranks(p):
        pfl = (p > 0).astype(jnp.float32)
        incl = (jnp.dot(pfl, Lm, preferred_element_type=jnp.float32)
                + jnp.dot(SL, jnp.dot(pfl, Jm,
                                      preferred_element_type=jnp.float32),
                          preferred_element_type=jnp.float32))
        return jnp.where(pfl > 0, incl - pfl, -1.0).astype(jnp.int32)

    nif_ref[...] = ranks(pf_ref[0] + pf_ref[1])
    nir_ref[...] = ranks(pr_ref[...])


_k7 = pl.pallas_call(
    _k7_body,
    in_specs=[pl.BlockSpec((2, 80, _C), lambda: (0, 0, 0)),
              pl.BlockSpec((80, _C), lambda: (0, 0))],
    out_specs=(pl.BlockSpec((80, _C), lambda: (0, 0)),
               pl.BlockSpec((80, _C), lambda: (0, 0))),
    out_shape=(jax.ShapeDtypeStruct((80, _C), jnp.int32),
               jax.ShapeDtypeStruct((80, _C), jnp.int32)),
)


_NCH = 400
_ECH = _E // 32
_RCH = _R // 32


@functools.partial(
    pl.kernel, mesh=_MESH, compiler_params=_SCP,
    out_type=(jax.ShapeDtypeStruct((_N, _C), jnp.float32),
              jax.ShapeDtypeStruct((_N, _C), jnp.float32),
              jax.ShapeDtypeStruct((_N,), jnp.int32),
              jax.ShapeDtypeStruct((_N,), jnp.int32),
              jax.ShapeDtypeStruct((2, _R), jnp.int32),
              jax.ShapeDtypeStruct((2, _E), jnp.int32)),
    scratch_types=[
        pltpu.VMEM((_NCH,), jnp.int32),
        pltpu.VMEM((_NCH,), jnp.int32),
        pltpu.VMEM((_NCH,), jnp.int32),
        pltpu.VMEM((_NCH, _C), jnp.float32),
        pltpu.VMEM((_ECH,), jnp.int32),
        pltpu.VMEM((_ECH,), jnp.int32),
        pltpu.VMEM((_RCH,), jnp.int32),
        pltpu.VMEM((_RCH,), jnp.int32),
        pltpu.VMEM((_N + 16,), jnp.int32),
        pltpu.VMEM_SHARED((_N + 16,), jnp.int32),
        pltpu.VMEM_SHARED((_N + 16,), jnp.int32),
        pltpu.VMEM_SHARED((_N,), jnp.int32),
        pltpu.VMEM_SHARED((_N,), jnp.int32),
        pltpu.VMEM_SHARED((_N,), jnp.int32),
        pltpu.SemaphoreType.DMA,
    ],
)
def _k8(h_hbm, batch_hbm, nif_hbm, nir_hbm, rsrc_hbm, rdst_hbm, src_hbm,
        dst_hbm, rx_hbm, fx_hbm, rb_hbm, fb_hbm, rei_hbm, fei_hbm,
        nv, posv, valv, rowsv, eb, ob, eb7, ob7, big, sub_f, sub_r, nif_sh,
        nir_sh, batch_sh, sem):
    cid = lax.axis_index("c")
    sid = lax.axis_index("s")
    w = _wid()
    lanes = _LANES()
    @pl.when(sid == 0)
    def _():
        pltpu.sync_copy(nif_hbm, big.at[pl.ds(0, _N)])
        pltpu.sync_copy(big.at[pl.ds(0, _N)], nif_sh)

    @pl.when(sid == 1)
    def _():
        pltpu.sync_copy(nir_hbm, big.at[pl.ds(0, _N)])
        pltpu.sync_copy(big.at[pl.ds(0, _N)], nir_sh)

    @pl.when(sid == 2)
    def _():
        pltpu.sync_copy(batch_hbm, big.at[pl.ds(0, _N)])
        pltpu.sync_copy(big.at[pl.ds(0, _N)], batch_sh)

    @pl.when(sid == 3)
    def _():
        _zero_vmem(big, _N + 16, jnp.int32)
        pltpu.sync_copy(big, sub_f)

    @pl.when(sid == 4)
    def _():
        _zero_vmem(big, _N + 16, jnp.int32)
        pltpu.sync_copy(big, sub_r)

    plsc.subcore_barrier()

    for br in range(2):
        ni_sh = nif_sh if br == 0 else nir_sh
        sub_sh = sub_f if br == 0 else sub_r
        for c in range(_N // _NCH):
            @pl.when(c % 16 == sid)
            def _(c=c, ni_sh=ni_sh, sub_sh=sub_sh):
                pltpu.sync_copy(ni_sh.at[pl.ds(c * _NCH, _NCH)], nv)

                def bld(i, cc):
                    pos = nv[pl.ds(i * 16, 16)]
                    posv[pl.ds(i * 16, 16)] = jnp.where(
                        pos < 0, _N + lanes, pos)
                    valv[pl.ds(i * 16, 16)] = c * _NCH + i * 16 + lanes
                    return cc
                lax.fori_loop(0, _NCH // 16, bld, 0)
                pltpu.sync_copy(valv, sub_sh.at[posv])
    plsc.subcore_barrier()

    for t in range(2):
        cidx = w + 32 * t

        @pl.when(cidx < _N // _NCH)
        def _(cidx=cidx):
            off = cidx * _NCH
            pltpu.sync_copy(sub_r.at[pl.ds(off, _NCH)], nv)
            pltpu.async_copy(h_hbm.at[nv], rowsv, sem).wait()
            pltpu.sync_copy(rowsv, rx_hbm.at[pl.ds(off, _NCH)])
            pltpu.sync_copy(batch_sh.at[nv], posv)
            pltpu.sync_copy(posv, rb_hbm.at[pl.ds(off, _NCH)])

        @pl.when(jnp.logical_and(cidx >= _N // _NCH,
                                 cidx < 2 * (_N // _NCH)))
        def _(cidx=cidx):
            off = (cidx - _N // _NCH) * _NCH
            pltpu.sync_copy(sub_f.at[pl.ds(off, _NCH)], nv)
            pltpu.async_copy(h_hbm.at[nv], rowsv, sem).wait()
            pltpu.sync_copy(rowsv, fx_hbm.at[pl.ds(off, _NCH)])
            pltpu.sync_copy(batch_sh.at[nv], posv)
            pltpu.sync_copy(posv, fb_hbm.at[pl.ds(off, _NCH)])

    eoff = w * _ECH
    pltpu.sync_copy(src_hbm.at[pl.ds(eoff, _ECH)], eb)
    pltpu.sync_copy(nif_sh.at[eb], ob)
    pltpu.sync_copy(ob, fei_hbm.at[0, pl.ds(eoff, _ECH)])
    pltpu.sync_copy(dst_hbm.at[pl.ds(eoff, _ECH)], eb)
    pltpu.sync_copy(nif_sh.at[eb], ob)
    pltpu.sync_copy(ob, fei_hbm.at[1, pl.ds(eoff, _ECH)])
    roff = w * _RCH
    pltpu.sync_copy(rsrc_hbm.at[pl.ds(roff, _RCH)], eb7)
    pltpu.sync_copy(nir_sh.at[eb7], ob7)
    pltpu.sync_copy(ob7, rei_hbm.at[0, pl.ds(roff, _RCH)])
    pltpu.sync_copy(rdst_hbm.at[pl.ds(roff, _RCH)], eb7)
    pltpu.sync_copy(nir_sh.at[eb7], ob7)
    pltpu.sync_copy(ob7, rei_hbm.at[1, pl.ds(roff, _RCH)])


def kernel(x, edge_index, edge_attr, batch, edge_score, c1_w1, c1_b1, c1_w2,
           c1_w3, c1_b3, c2_w1, c2_b1, c2_w2, c2_w3, c2_b3):
    src = edge_index[0]
    dst = edge_index[1]
    ew = edge_attr.reshape(-1)
    xe = jnp.concatenate(
        [x, jnp.ones((_N, 1), jnp.float32), jnp.zeros((_N, 11), jnp.float32)],
        axis=1)

    acc1, pres_f = _k1(xe, edge_index, ew)

    zw = jnp.zeros((16, _C), jnp.float32)
    w1a = zw.at[0:4].set(c1_w1).at[4].set(c1_b1)
    w2a = zw.at[0:4].set(c1_w2)
    w3a = zw.at[0:4].set(c1_w3).at[4].set(c1_b3)
    h1 = _k2(acc1, xe, w1a, w2a, w3a)

    acc2 = _k3(h1, edge_index, ew)
    h = _k4(acc2, acc1, h1, c2_w1, c2_w2, c2_w3, c2_b1[None, :],
            c2_b3[None, :])

    r_ew, r_ea, r_src, r_dst, pres_r = _k5(edge_score, ew, src, dst)

    pf = jnp.pad(pres_f, ((0, 0), (0, 10240 - _N))).reshape(2, 80, _C)
    pr = jnp.pad(pres_r, (0, 10240 - _N)).reshape(80, _C)
    nif80, nir80 = _k7(pf, pr)
    nif = nif80.reshape(-1)[:_N]
    nir = nir80.reshape(-1)[:_N]

    r_x, f_x, r_batch, f_batch, r_ei2, f_ei2 = _k8(
        h, batch, nif, nir, r_src, r_dst, src, dst)

    return ((r_x, r_ei2, r_ea, r_ew, r_batch),
            (f_x, f_ei2, ew, edge_score, f_batch),
            edge_score)


# speedup vs baseline: 17.5701x; 1.1253x over previous
import functools
import jax
import jax.numpy as jnp
from jax import lax
from jax.experimental import pallas as pl
from jax.experimental.pallas import tpu as pltpu, tpu_sc as plsc

_N = 10000
_E = 320000
_C = 128
_R = 224000
_KEYMAX = 0x3F7FFFFF

_MESH = plsc.VectorSubcoreMesh(core_axis_name="c", subcore_axis_name="s")
_SCP = pltpu.CompilerParams(needs_layout_passes=False, use_tc_tiling_on_sc=False)

_LANES = lambda: lax.iota(jnp.int32, 16)
_ZI = lambda: jnp.zeros((16,), jnp.int32)
_ZF = lambda: jnp.zeros((16,), jnp.float32)


def _wid():
    return lax.axis_index("c") * 16 + lax.axis_index("s")


def _zero_vmem(ref, n, dtype):
    z = jnp.zeros((16,), dtype)

    def body(i, c):
        ref[pl.ds(i * 16, 16)] = z
        return c

    lax.fori_loop(0, n // 16, body, 0)


_CH1 = 1000


@functools.partial(
    pl.kernel, mesh=_MESH, compiler_params=_SCP,
    out_type=(jax.ShapeDtypeStruct((2, _N, 16), jnp.float32),
              jax.ShapeDtypeStruct((2, _N), jnp.float32)),
    scratch_types=[
        pltpu.VMEM((_CH1,), jnp.int32),
        pltpu.VMEM((_CH1,), jnp.int32),
        pltpu.VMEM((_CH1,), jnp.float32),
        pltpu.VMEM((_CH1, 16), jnp.float32),
        pltpu.VMEM((_CH1,), jnp.float32),
        pltpu.VMEM((640, 16), jnp.float32),
        pltpu.VMEM((640,), jnp.float32),
        pltpu.VMEM_SHARED((_N, 16), jnp.float32),
        pltpu.VMEM_SHARED((_N,), jnp.float32),
        pltpu.SemaphoreType.DMA,
    ],
)
def _k1(xe_hbm, ei_hbm, ew_hbm, acc_hbm, pres_hbm,
        srcv, dstv, ewv, rowsv, onesv, zrows, zi640, acc_sh, pres_sh, sem):
    cid = lax.axis_index("c")
    sid = lax.axis_index("s")
    w = _wid()
    lanes = _LANES()
    def zr(i, c):
        zrows[i, :] = _ZF()
        return c
    lax.fori_loop(0, 640, zr, 0)
    _zero_vmem(zi640, 640, jnp.float32)
    pltpu.sync_copy(zrows.at[pl.ds(0, 625)], acc_sh.at[pl.ds(sid * 625, 625)])

    @pl.when(sid < 15)
    def _():
        pltpu.sync_copy(zi640, pres_sh.at[pl.ds(sid * 640, 640)])

    @pl.when(sid == 15)
    def _():
        pltpu.sync_copy(zi640.at[pl.ds(0, 400)], pres_sh.at[pl.ds(9600, 400)])

    def o1(i, c):
        onesv[pl.ds(i * 16, 16)] = _ZF() + 1.0
        return c
    lax.fori_loop(0, _CH1 // 16, o1, 0)
    plsc.subcore_barrier()

    base_w = w * (_E // 32)
    for j in range((_E // 32) // _CH1):
        off = base_w + j * _CH1
        pltpu.sync_copy(ei_hbm.at[0, pl.ds(off, _CH1)], srcv)
        pltpu.sync_copy(ei_hbm.at[1, pl.ds(off, _CH1)], dstv)
        pltpu.sync_copy(ew_hbm.at[pl.ds(off, _CH1)], ewv)
        pltpu.async_copy(xe_hbm.at[srcv], rowsv, sem).wait()

        def mul(r, c):
            s = plsc.load_gather(ewv, [_ZI() + r])
            rowsv[r, :] = rowsv[r, :] * s
            return c

        lax.fori_loop(0, _CH1, mul, 0)
        pltpu.sync_copy(rowsv, acc_sh.at[dstv], add=True)
        pltpu.sync_copy(onesv, pres_sh.at[srcv], add=True)
        pltpu.sync_copy(onesv, pres_sh.at[dstv], add=True)

    plsc.subcore_barrier()
    pltpu.sync_copy(acc_sh.at[pl.ds(sid * 625, 625)],
                    acc_hbm.at[cid, pl.ds(sid * 625, 625)])

    @pl.when(sid < 15)
    def _():
        pltpu.sync_copy(pres_sh.at[pl.ds(sid * 640, 640)],
                        pres_hbm.at[cid, pl.ds(sid * 640, 640)])

    @pl.when(sid == 15)
    def _():
        pltpu.sync_copy(pres_sh.at[pl.ds(9600, 400)],
                        pres_hbm.at[cid, pl.ds(9600, 400)])


_CH3 = 160
_NFULL3 = (_E // 32) // _CH3
_TAIL3 = (_E // 32) - _NFULL3 * _CH3


@functools.partial(
    pl.kernel, mesh=_MESH, compiler_params=_SCP,
    out_type=jax.ShapeDtypeStruct((2, _N, _C), jnp.float32),
    scratch_types=[
        pltpu.VMEM((_CH3,), jnp.int32),
        pltpu.VMEM((_CH3,), jnp.int32),
        pltpu.VMEM((_CH3,), jnp.float32),
        pltpu.VMEM((_CH3, _C), jnp.float32),
        pltpu.VMEM((_CH3,), jnp.int32),
        pltpu.VMEM((_CH3,), jnp.int32),
        pltpu.VMEM((_CH3,), jnp.float32),
        pltpu.VMEM((_CH3, _C), jnp.float32),
        pltpu.VMEM_SHARED((_N, _C), jnp.float32),
        pltpu.SemaphoreType.DMA,
        pltpu.SemaphoreType.DMA,
    ],
)
def _k3(h1_hbm, ei_hbm, ew_hbm, acc_hbm,
        srcv0, dstv0, ewv0, rowsv0, srcv1, dstv1, ewv1, rowsv1,
        acc_sh, sem0, sem1):
    cid = lax.axis_index("c")
    sid = lax.axis_index("s")
    w = _wid()
    def zr(i, c):
        for jj in range(8):
            rowsv0[i, pl.ds(jj * 16, 16)] = _ZF()
        return c
    lax.fori_loop(0, _CH3, zr, 0)
    for z in range(3):
        pltpu.sync_copy(rowsv0, acc_sh.at[pl.ds(sid * 625 + z * _CH3, _CH3)])
    pltpu.sync_copy(rowsv0.at[pl.ds(0, 625 - 3 * _CH3)],
                    acc_sh.at[pl.ds(sid * 625 + 3 * _CH3, 625 - 3 * _CH3)])
    plsc.subcore_barrier()

    base_w = w * (_E // 32)

    def load_idx(off, srcv, dstv, ewv, n=_CH3):
        pltpu.sync_copy(ei_hbm.at[0, pl.ds(off, n)], srcv.at[pl.ds(0, n)])
        pltpu.sync_copy(ei_hbm.at[1, pl.ds(off, n)], dstv.at[pl.ds(0, n)])
        pltpu.sync_copy(ew_hbm.at[pl.ds(off, n)], ewv.at[pl.ds(0, n)])

    def compute_scatter(dstv, ewv, rowsv):
        def mul(r, c):
            sc = plsc.load_gather(ewv, [_ZI() + r])
            for jj in range(8):
                rowsv[r, pl.ds(jj * 16, 16)] = (
                    rowsv[r, pl.ds(jj * 16, 16)] * sc)
            return c
        lax.fori_loop(0, _CH3, mul, 0)
        pltpu.sync_copy(rowsv, acc_sh.at[dstv], add=True)

    load_idx(base_w, srcv0, dstv0, ewv0)
    g0 = pltpu.async_copy(h1_hbm.at[srcv0], rowsv0, sem0)

    def body(jj, c):
        off1 = base_w + (2 * jj + 1) * _CH3
        load_idx(off1, srcv1, dstv1, ewv1)
        g1 = pltpu.async_copy(h1_hbm.at[srcv1], rowsv1, sem1)
        pltpu.make_async_copy(h1_hbm.at[srcv0], rowsv0, sem0).wait()
        compute_scatter(dstv0, ewv0, rowsv0)

        @pl.when(jj < _NFULL3 // 2 - 1)
        def _():
            off2 = base_w + (2 * jj + 2) * _CH3
            load_idx(off2, srcv0, dstv0, ewv0)
            pltpu.async_copy(h1_hbm.at[srcv0], rowsv0, sem0)
        g1.wait()
        compute_scatter(dstv1, ewv1, rowsv1)
        return c

    lax.fori_loop(0, _NFULL3 // 2, body, 0)

    toff = base_w + _NFULL3 * _CH3
    load_idx(toff, srcv0, dstv0, ewv0, n=_TAIL3)
    pltpu.async_copy(h1_hbm.at[srcv0.at[pl.ds(0, _TAIL3)]],
                     rowsv0.at[pl.ds(0, _TAIL3)], sem0).wait()

    def mult(r, c):
        sc = plsc.load_gather(ewv0, [_ZI() + r])
        for jj in range(8):
            rowsv0[r, pl.ds(jj * 16, 16)] = rowsv0[r, pl.ds(jj * 16, 16)] * sc
        return c
    lax.fori_loop(0, _TAIL3, mult, 0)
    pltpu.sync_copy(rowsv0.at[pl.ds(0, _TAIL3)],
                    acc_sh.at[dstv0.at[pl.ds(0, _TAIL3)]], add=True)

    plsc.subcore_barrier()
    pltpu.sync_copy(acc_sh.at[pl.ds(sid * 625, 625)],
                    acc_hbm.at[cid, pl.ds(sid * 625, 625)])


def _k2_body(acc_ref, x_ref, w1_ref, w2_ref, w3_ref, h1_ref):
    acc = acc_ref[0] + acc_ref[1]
    deg = acc[:, 4:5]
    x = x_ref[...]
    t1 = jnp.dot(acc, w1_ref[...], preferred_element_type=jnp.float32)
    t2 = jnp.dot(x, w2_ref[...], preferred_element_type=jnp.float32)
    t3 = jnp.dot(x, w3_ref[...], preferred_element_type=jnp.float32)
    h1_ref[...] = jnp.maximum(t1 - deg * t2 + t3, 0.0)


_k2 = pl.pallas_call(
    _k2_body,
    grid=(5,),
    in_specs=[
        pl.BlockSpec((2, 2000, 16), lambda i: (0, i, 0)),
        pl.BlockSpec((2000, 16), lambda i: (i, 0)),
        pl.BlockSpec((16, _C), lambda i: (0, 0)),
        pl.BlockSpec((16, _C), lambda i: (0, 0)),
        pl.BlockSpec((16, _C), lambda i: (0, 0)),
    ],
    out_specs=pl.BlockSpec((2000, _C), lambda i: (i, 0)),
    out_shape=jax.ShapeDtypeStruct((_N, _C), jnp.float32),
)


def _k4_body(acc2_ref, acc1_ref, h1_ref, w1_ref, w2_ref, w3_ref, b1_ref,
             b3_ref, h_ref):
    z2 = acc2_ref[0] + acc2_ref[1]
    deg = acc1_ref[0][:, 4:5] + acc1_ref[1][:, 4:5]
    h1 = h1_ref[...]
    t1 = jnp.dot(z2, w1_ref[...], preferred_element_type=jnp.float32)
    t2 = jnp.dot(h1, w2_ref[...], preferred_element_type=jnp.float32)
    t3 = jnp.dot(h1, w3_ref[...], preferred_element_type=jnp.float32)
    h_ref[...] = t1 + deg * b1_ref[...] - deg * t2 + t3 + b3_ref[...]


_k4 = pl.pallas_call(
    _k4_body,
    grid=(5,),
    in_specs=[
        pl.BlockSpec((2, 2000, _C), lambda i: (0, i, 0)),
        pl.BlockSpec((2, 2000, 16), lambda i: (0, i, 0)),
        pl.BlockSpec((2000, _C), lambda i: (i, 0)),
        pl.BlockSpec((_C, _C), lambda i: (0, 0)),
        pl.BlockSpec((_C, _C), lambda i: (0, 0)),
        pl.BlockSpec((_C, _C), lambda i: (0, 0)),
        pl.BlockSpec((1, _C), lambda i: (0, 0)),
        pl.BlockSpec((1, _C), lambda i: (0, 0)),
    ],
    out_specs=pl.BlockSpec((2000, _C), lambda i: (i, 0)),
    out_shape=jax.ShapeDtypeStruct((_N, _C), jnp.float32),
)


_W5 = _E // 16
_S5 = _W5 // 16
_RDX = 256
_KCH = 2000


@functools.partial(
    pl.kernel, mesh=_MESH, compiler_params=_SCP,
    out_type=(jax.ShapeDtypeStruct((_R,), jnp.float32),
              jax.ShapeDtypeStruct((_R,), jnp.float32),
              jax.ShapeDtypeStruct((_R,), jnp.int32),
              jax.ShapeDtypeStruct((_R,), jnp.int32),
              jax.ShapeDtypeStruct((_N,), jnp.float32)),
    scratch_types=[
        pltpu.VMEM((_W5,), jnp.float32),
        pltpu.VMEM((_W5,), jnp.int32),
        pltpu.VMEM((_W5,), jnp.int32),
        pltpu.VMEM((16 * _RDX,), jnp.int32),
        pltpu.VMEM((16 * _RDX,), jnp.int32),
        pltpu.VMEM((_RDX,), jnp.int32),
        pltpu.VMEM((_KCH,), jnp.float32),
        pltpu.VMEM((_KCH,), jnp.int32),
        pltpu.VMEM((_KCH,), jnp.int32),
        pltpu.VMEM((_KCH,), jnp.float32),
        pltpu.VMEM((640,), jnp.float32),
        pltpu.VMEM_SHARED((_E,), jnp.int32),
        pltpu.VMEM_SHARED((_E,), jnp.int32),
        pltpu.VMEM_SHARED((16 * _RDX,), jnp.int32),
        pltpu.VMEM_SHARED((_N,), jnp.float32),
        pltpu.SemaphoreType.DMA,
    ],
)
def _k5(score_hbm, ew_hbm, src_hbm, dst_hbm,
        rew_hbm, rea_hbm, rsrc_hbm, rdst_hbm, presr_hbm,
        keysf, valsv, posv, histv, Tallv, Tv, s2k, i2k, j2k, ones2k,
        zi640, A_v, B_v, Tall_sh, pres_r_sh, sem):
    cid = lax.axis_index("c")
    sid = lax.axis_index("s")
    lanes = _LANES()

    @pl.when(cid == 0)
    def _():
        base = sid * _W5
        _zero_vmem(zi640, 640, jnp.float32)

        @pl.when(sid < 15)
        def _():
            pltpu.sync_copy(zi640, pres_r_sh.at[pl.ds(sid * 640, 640)])

        @pl.when(sid == 15)
        def _():
            pltpu.sync_copy(zi640.at[pl.ds(0, 400)],
                            pres_r_sh.at[pl.ds(9600, 400)])

        def o1(i, c):
            ones2k[pl.ds(i * 16, 16)] = _ZF() + 1.0
            return c
        lax.fori_loop(0, _KCH // 16, o1, 0)

        for p in range(4):
            shift = 8 * p
            if p % 2 == 0:
                src_v, dst_v = A_v, B_v
            else:
                src_v, dst_v = B_v, A_v
            if p == 0:
                pltpu.sync_copy(score_hbm.at[pl.ds(base, _W5)], keysf)

                def iv(i, c):
                    valsv[pl.ds(i * 16, 16)] = base + i * 16 + lanes
                    return c
                lax.fori_loop(0, _W5 // 16, iv, 0)
            else:
                pltpu.sync_copy(src_v.at[pl.ds(base, _W5)], valsv)
                pltpu.async_copy(score_hbm.at[valsv], keysf, sem).wait()

            def mk(i, c):
                sl = pl.ds(i * 16, 16)
                keysf[sl] = plsc.bitcast(
                    _KEYMAX - plsc.bitcast(keysf[sl], jnp.int32), jnp.float32)
                return c
            lax.fori_loop(0, _W5 // 16, mk, 0)

            def zh(i, c):
                histv[pl.ds(i * 16, 16)] = _ZI()
                return c
            lax.fori_loop(0, _RDX, zh, 0)

            def ha(t, c):
                sl = lanes * _S5 + t
                kv = plsc.bitcast(plsc.load_gather(keysf, [sl]), jnp.int32)
                d = jnp.bitwise_and(lax.shift_right_logical(kv, shift),
                                    _RDX - 1)
                fl = lanes * _RDX + d
                cnt = plsc.load_gather(histv, [fl])
                plsc.store_scatter(histv, [fl], cnt + 1)
                plsc.store_scatter(posv, [sl], cnt)
                return c
            lax.fori_loop(0, _S5, ha, 0)

            def wt(j, c):
                acc = _ZI()
                for l in range(16):
                    acc = acc + histv[pl.ds(l * _RDX + j * 16, 16)]
                Tv[pl.ds(j * 16, 16)] = acc
                return c
            lax.fori_loop(0, _RDX // 16, wt, 0)
            pltpu.sync_copy(Tv, Tall_sh.at[pl.ds(sid * _RDX, _RDX)])
            plsc.subcore_barrier()

            pltpu.sync_copy(Tall_sh, Tallv)

            def pb(j, carry):
                G = _ZI()
                Wp = _ZI()
                for wk in range(16):
                    s_w = Tallv[pl.ds(wk * _RDX + j * 16, 16)]
                    G = G + s_w
                    Wp = Wp + s_w * jnp.where(wk < sid, 1, 0)
                cs = plsc.cumsum(G)
                Px = carry + cs - G
                run = Px + Wp
                for l in range(16):
                    sl_ = pl.ds(l * _RDX + j * 16, 16)
                    tmp = histv[sl_]
                    histv[sl_] = run
                    run = run + tmp
                return carry + jnp.sum(G)
            lax.fori_loop(0, _RDX // 16, pb, jnp.int32(0))

            def pc(t, c):
                sl = lanes * _S5 + t
                kv = plsc.bitcast(plsc.load_gather(keysf, [sl]), jnp.int32)
                d = jnp.bitwise_and(lax.shift_right_logical(kv, shift),
                                    _RDX - 1)
                base_off = plsc.load_gather(histv, [lanes * _RDX + d])
                ordn = plsc.load_gather(posv, [sl])
                plsc.store_scatter(posv, [sl], base_off + ordn)
                return c
            lax.fori_loop(0, _S5, pc, 0)

            pltpu.sync_copy(valsv, dst_v.at[posv])
            plsc.subcore_barrier()

        base2 = sid * (_R // 16)
        for q in range((_R // 16) // _KCH):
            off2 = base2 + q * _KCH
            pltpu.sync_copy(A_v.at[pl.ds(off2, _KCH)], i2k)
            pltpu.async_copy(score_hbm.at[i2k], s2k, sem).wait()
            pltpu.sync_copy(s2k, rew_hbm.at[pl.ds(off2, _KCH)])
            pltpu.async_copy(ew_hbm.at[i2k], s2k, sem).wait()
            pltpu.sync_copy(s2k, rea_hbm.at[pl.ds(off2, _KCH)])
            pltpu.async_copy(src_hbm.at[i2k], j2k, sem).wait()
            pltpu.sync_copy(j2k, rsrc_hbm.at[pl.ds(off2, _KCH)])
            pltpu.sync_copy(ones2k, pres_r_sh.at[j2k], add=True)
            pltpu.async_copy(dst_hbm.at[i2k], j2k, sem).wait()
            pltpu.sync_copy(j2k, rdst_hbm.at[pl.ds(off2, _KCH)])
            pltpu.sync_copy(ones2k, pres_r_sh.at[j2k], add=True)

        plsc.subcore_barrier()

        @pl.when(sid < 15)
        def _():
            pltpu.sync_copy(pres_r_sh.at[pl.ds(sid * 640, 640)],
                            presr_hbm.at[pl.ds(sid * 640, 640)])

        @pl.when(sid == 15)
        def _():
            pltpu.sync_copy(pres_r_sh.at[pl.ds(9600, 400)],
                            presr_hbm.at[pl.ds(9600, 400)])


def _k7_body(pf_ref, pr_ref, nif_ref, nir_ref):
    r128 = lax.broadcasted_iota(jnp.int32, (_C, _C), 0)
    c128 = lax.broadcasted_iota(jnp.int32, (_C, _C), 1)
    Lm = (r128 <= c128).astype(jnp.float32)
    Jm = jnp.ones((_C, _C), jnp.float32)
    r80 = lax.broadcasted_iota(jnp.int32, (80, 80), 0)
    c80 = lax.broadcasted_iota(jnp.int32, (80, 80), 1)
    SL = (c80 < r80).astype(jnp.float32)

    def ranks(p):
        pfl = (p > 0).astype(jnp.float32)
        incl = (jnp.dot(pfl, Lm, preferred_element_type=jnp.float32)
                + jnp.dot(SL, jnp.dot(pfl, Jm,
                                      preferred_element_type=jnp.float32),
                          preferred_element_type=jnp.float32))
        return jnp.where(pfl > 0, incl - pfl, -1.0).astype(jnp.int32)

    nif_ref[...] = ranks(pf_ref[0] + pf_ref[1])
    nir_ref[...] = ranks(pr_ref[...])


_k7 = pl.pallas_call(
    _k7_body,
    in_specs=[pl.BlockSpec((2, 80, _C), lambda: (0, 0, 0)),
              pl.BlockSpec((80, _C), lambda: (0, 0))],
    out_specs=(pl.BlockSpec((80, _C), lambda: (0, 0)),
               pl.BlockSpec((80, _C), lambda: (0, 0))),
    out_shape=(jax.ShapeDtypeStruct((80, _C), jnp.int32),
               jax.ShapeDtypeStruct((80, _C), jnp.int32)),
)


_NCH = 400
_ECH = _E // 32
_RCH = _R // 32


@functools.partial(
    pl.kernel, mesh=_MESH, compiler_params=_SCP,
    out_type=(jax.ShapeDtypeStruct((_N, _C), jnp.float32),
              jax.ShapeDtypeStruct((_N, _C), jnp.float32),
              jax.ShapeDtypeStruct((_N,), jnp.int32),
              jax.ShapeDtypeStruct((_N,), jnp.int32),
              jax.ShapeDtypeStruct((2, _R), jnp.int32),
              jax.ShapeDtypeStruct((2, _E), jnp.int32)),
    scratch_types=[
        pltpu.VMEM((_NCH,), jnp.int32),
        pltpu.VMEM((_NCH,), jnp.int32),
        pltpu.VMEM((_NCH,), jnp.int32),
        pltpu.VMEM((_NCH, _C), jnp.float32),
        pltpu.VMEM((_ECH,), jnp.int32),
        pltpu.VMEM((_ECH,), jnp.int32),
        pltpu.VMEM((_RCH,), jnp.int32),
        pltpu.VMEM((_RCH,), jnp.int32),
        pltpu.VMEM((_N + 16,), jnp.int32),
        pltpu.VMEM_SHARED((_N + 16,), jnp.int32),
        pltpu.VMEM_SHARED((_N + 16,), jnp.int32),
        pltpu.VMEM_SHARED((_N,), jnp.int32),
        pltpu.VMEM_SHARED((_N,), jnp.int32),
        pltpu.VMEM_SHARED((_N,), jnp.int32),
        pltpu.SemaphoreType.DMA,
    ],
)
def _k8(h_hbm, batch_hbm, nif_hbm, nir_hbm, rsrc_hbm, rdst_hbm, src_hbm,
        dst_hbm, rx_hbm, fx_hbm, rb_hbm, fb_hbm, rei_hbm, fei_hbm,
        nv, posv, valv, rowsv, eb, ob, eb7, ob7, big, sub_f, sub_r, nif_sh,
        nir_sh, batch_sh, sem):
    cid = lax.axis_index("c")
    sid = lax.axis_index("s")
    w = _wid()
    lanes = _LANES()
    @pl.when(sid == 0)
    def _():
        pltpu.sync_copy(nif_hbm, big.at[pl.ds(0, _N)])
        pltpu.sync_copy(big.at[pl.ds(0, _N)], nif_sh)

    @pl.when(sid == 1)
    def _():
        pltpu.sync_copy(nir_hbm, big.at[pl.ds(0, _N)])
        pltpu.sync_copy(big.at[pl.ds(0, _N)], nir_sh)

    @pl.when(sid == 2)
    def _():
        pltpu.sync_copy(batch_hbm, big.at[pl.ds(0, _N)])
        pltpu.sync_copy(big.at[pl.ds(0, _N)], batch_sh)

    @pl.when(sid == 3)
    def _():
        _zero_vmem(big, _N + 16, jnp.int32)
        pltpu.sync_copy(big, sub_f)

    @pl.when(sid == 4)
    def _():
        _zero_vmem(big, _N + 16, jnp.int32)
        pltpu.sync_copy(big, sub_r)

    plsc.subcore_barrier()

    for br in range(2):
        ni_sh = nif_sh if br == 0 else nir_sh
        sub_sh = sub_f if br == 0 else sub_r
        for c in range(_N // _NCH):
            @pl.when(c % 16 == sid)
            def _(c=c, ni_sh=ni_sh, sub_sh=sub_sh):
                pltpu.sync_copy(ni_sh.at[pl.ds(c * _NCH, _NCH)], nv)

                def bld(i, cc):
                    pos = nv[pl.ds(i * 16, 16)]
                    posv[pl.ds(i * 16, 16)] = jnp.where(
                        pos < 0, _N + lanes, pos)
                    valv[pl.ds(i * 16, 16)] = c * _NCH + i * 16 + lanes
                    return cc
                lax.fori_loop(0, _NCH // 16, bld, 0)
                pltpu.sync_copy(valv, sub_sh.at[posv])
    plsc.subcore_barrier()

    for t in range(2):
        cidx = w + 32 * t

        @pl.when(cidx < _N // _NCH)
        def _(cidx=cidx):
            off = cidx * _NCH
            pltpu.sync_copy(sub_r.at[pl.ds(off, _NCH)], nv)
            pltpu.async_copy(h_hbm.at[nv], rowsv, sem).wait()
            pltpu.sync_copy(rowsv, rx_hbm.at[pl.ds(off, _NCH)])
            pltpu.sync_copy(batch_sh.at[nv], posv)
            pltpu.sync_copy(posv, rb_hbm.at[pl.ds(off, _NCH)])

        @pl.when(jnp.logical_and(cidx >= _N // _NCH,
                                 cidx < 2 * (_N // _NCH)))
        def _(cidx=cidx):
            off = (cidx - _N // _NCH) * _NCH
            pltpu.sync_copy(sub_f.at[pl.ds(off, _NCH)], nv)
            pltpu.async_copy(h_hbm.at[nv], rowsv, sem).wait()
            pltpu.sync_copy(rowsv, fx_hbm.at[pl.ds(off, _NCH)])
            pltpu.sync_copy(batch_sh.at[nv], posv)
            pltpu.sync_copy(posv, fb_hbm.at[pl.ds(off, _NCH)])

    eoff = w * _ECH
    pltpu.sync_copy(src_hbm.at[pl.ds(eoff, _ECH)], eb)
    pltpu.sync_copy(nif_sh.at[eb], ob)
    pltpu.sync_copy(ob, fei_hbm.at[0, pl.ds(eoff, _ECH)])
    pltpu.sync_copy(dst_hbm.at[pl.ds(eoff, _ECH)], eb)
    pltpu.sync_copy(nif_sh.at[eb], ob)
    pltpu.sync_copy(ob, fei_hbm.at[1, pl.ds(eoff, _ECH)])
    roff = w * _RCH
    pltpu.sync_copy(rsrc_hbm.at[pl.ds(roff, _RCH)], eb7)
    pltpu.sync_copy(nir_sh.at[eb7], ob7)
    pltpu.sync_copy(ob7, rei_hbm.at[0, pl.ds(roff, _RCH)])
    pltpu.sync_copy(rdst_hbm.at[pl.ds(roff, _RCH)], eb7)
    pltpu.sync_copy(nir_sh.at[eb7], ob7)
    pltpu.sync_copy(ob7, rei_hbm.at[1, pl.ds(roff, _RCH)])


def kernel(x, edge_index, edge_attr, batch, edge_score, c1_w1, c1_b1, c1_w2,
           c1_w3, c1_b3, c2_w1, c2_b1, c2_w2, c2_w3, c2_b3):
    src = edge_index[0]
    dst = edge_index[1]
    ew = edge_attr.reshape(-1)
    xe = jnp.concatenate(
        [x, jnp.ones((_N, 1), jnp.float32), jnp.zeros((_N, 11), jnp.float32)],
        axis=1)

    acc1, pres_f = _k1(xe, edge_index, ew)

    zw = jnp.zeros((16, _C), jnp.float32)
    w1a = zw.at[0:4].set(c1_w1).at[4].set(c1_b1)
    w2a = zw.at[0:4].set(c1_w2)
    w3a = zw.at[0:4].set(c1_w3).at[4].set(c1_b3)
    h1 = _k2(acc1, xe, w1a, w2a, w3a)

    acc2 = _k3(h1, edge_index, ew)
    h = _k4(acc2, acc1, h1, c2_w1, c2_w2, c2_w3, c2_b1[None, :],
            c2_b3[None, :])

    r_ew, r_ea, r_src, r_dst, pres_r = _k5(edge_score, ew, src, dst)

    pf = jnp.pad(pres_f, ((0, 0), (0, 10240 - _N))).reshape(2, 80, _C)
    pr = jnp.pad(pres_r, (0, 10240 - _N)).reshape(80, _C)
    nif80, nir80 = _k7(pf, pr)
    nif = nif80.reshape(-1)[:_N]
    nir = nir80.reshape(-1)[:_N]

    r_x, f_x, r_batch, f_batch, r_ei2, f_ei2 = _k8(
        h, batch, nif, nir, r_src, r_dst, src, dst)

    return ((r_x, r_ei2, r_ea, r_ew, r_batch),
            (f_x, f_ei2, ew, edge_score, f_batch),
            edge_score)


# speedup vs baseline: 18.0172x; 1.0254x over previous
import functools
import jax
import jax.numpy as jnp
from jax import lax
from jax.experimental import pallas as pl
from jax.experimental.pallas import tpu as pltpu, tpu_sc as plsc

_N = 10000
_E = 320000
_C = 128
_R = 224000
_KEYMAX = 0x3F7FFFFF

_MESH = plsc.VectorSubcoreMesh(core_axis_name="c", subcore_axis_name="s")
_SCP = pltpu.CompilerParams(needs_layout_passes=False, use_tc_tiling_on_sc=False)

_LANES = lambda: lax.iota(jnp.int32, 16)
_ZI = lambda: jnp.zeros((16,), jnp.int32)
_ZF = lambda: jnp.zeros((16,), jnp.float32)


def _wid():
    return lax.axis_index("c") * 16 + lax.axis_index("s")


def _zero_vmem(ref, n, dtype):
    z = jnp.zeros((16,), dtype)

    def body(i, c):
        ref[pl.ds(i * 16, 16)] = z
        return c

    lax.fori_loop(0, n // 16, body, 0)


_CH1 = 1000


@functools.partial(
    pl.kernel, mesh=_MESH, compiler_params=_SCP,
    out_type=(jax.ShapeDtypeStruct((2, _N, 16), jnp.float32),
              jax.ShapeDtypeStruct((2, _N), jnp.float32)),
    scratch_types=[
        pltpu.VMEM((_CH1,), jnp.int32),
        pltpu.VMEM((_CH1,), jnp.int32),
        pltpu.VMEM((_CH1,), jnp.float32),
        pltpu.VMEM((_CH1, 16), jnp.float32),
        pltpu.VMEM((_CH1,), jnp.int32),
        pltpu.VMEM((_CH1,), jnp.int32),
        pltpu.VMEM((_CH1,), jnp.float32),
        pltpu.VMEM((_CH1, 16), jnp.float32),
        pltpu.VMEM((_CH1,), jnp.float32),
        pltpu.VMEM((640, 16), jnp.float32),
        pltpu.VMEM((640,), jnp.float32),
        pltpu.VMEM_SHARED((_N, 16), jnp.float32),
        pltpu.VMEM_SHARED((_N,), jnp.float32),
        pltpu.SemaphoreType.DMA,
        pltpu.SemaphoreType.DMA,
    ],
)
def _k1(xe_hbm, ei_hbm, ew_hbm, acc_hbm, pres_hbm,
        srcv0, dstv0, ewv0, rowsv0, srcv1, dstv1, ewv1, rowsv1,
        onesv, zrows, zi640, acc_sh, pres_sh, sem0, sem1):
    cid = lax.axis_index("c")
    sid = lax.axis_index("s")
    w = _wid()
    lanes = _LANES()
    def zr(i, c):
        zrows[i, :] = _ZF()
        return c
    lax.fori_loop(0, 640, zr, 0)
    _zero_vmem(zi640, 640, jnp.float32)
    pltpu.sync_copy(zrows.at[pl.ds(0, 625)], acc_sh.at[pl.ds(sid * 625, 625)])

    @pl.when(sid < 15)
    def _():
        pltpu.sync_copy(zi640, pres_sh.at[pl.ds(sid * 640, 640)])

    @pl.when(sid == 15)
    def _():
        pltpu.sync_copy(zi640.at[pl.ds(0, 400)], pres_sh.at[pl.ds(9600, 400)])

    def o1(i, c):
        onesv[pl.ds(i * 16, 16)] = _ZF() + 1.0
        return c
    lax.fori_loop(0, _CH1 // 16, o1, 0)
    plsc.subcore_barrier()

    base_w = w * (_E // 32)
    _NC1 = (_E // 32) // _CH1

    def load_idx(off, srcv, dstv, ewv):
        pltpu.sync_copy(ei_hbm.at[0, pl.ds(off, _CH1)], srcv)
        pltpu.sync_copy(ei_hbm.at[1, pl.ds(off, _CH1)], dstv)
        pltpu.sync_copy(ew_hbm.at[pl.ds(off, _CH1)], ewv)

    def compute_scatter(srcv, dstv, ewv, rowsv):
        def mul(r, c):
            sc = plsc.load_gather(ewv, [_ZI() + r])
            rowsv[r, :] = rowsv[r, :] * sc
            return c
        lax.fori_loop(0, _CH1, mul, 0)
        pltpu.sync_copy(rowsv, acc_sh.at[dstv], add=True)
        pltpu.sync_copy(onesv, pres_sh.at[srcv], add=True)
        pltpu.sync_copy(onesv, pres_sh.at[dstv], add=True)

    load_idx(base_w, srcv0, dstv0, ewv0)
    pltpu.async_copy(xe_hbm.at[srcv0], rowsv0, sem0)

    def body(jj, c):
        off1 = base_w + (2 * jj + 1) * _CH1
        load_idx(off1, srcv1, dstv1, ewv1)
        g1 = pltpu.async_copy(xe_hbm.at[srcv1], rowsv1, sem1)
        pltpu.make_async_copy(xe_hbm.at[srcv0], rowsv0, sem0).wait()
        compute_scatter(srcv0, dstv0, ewv0, rowsv0)

        @pl.when(jj < _NC1 // 2 - 1)
        def _():
            off2 = base_w + (2 * jj + 2) * _CH1
            load_idx(off2, srcv0, dstv0, ewv0)
            pltpu.async_copy(xe_hbm.at[srcv0], rowsv0, sem0)
        g1.wait()
        compute_scatter(srcv1, dstv1, ewv1, rowsv1)
        return c

    lax.fori_loop(0, _NC1 // 2, body, 0)

    plsc.subcore_barrier()
    pltpu.sync_copy(acc_sh.at[pl.ds(sid * 625, 625)],
                    acc_hbm.at[cid, pl.ds(sid * 625, 625)])

    @pl.when(sid < 15)
    def _():
        pltpu.sync_copy(pres_sh.at[pl.ds(sid * 640, 640)],
                        pres_hbm.at[cid, pl.ds(sid * 640, 640)])

    @pl.when(sid == 15)
    def _():
        pltpu.sync_copy(pres_sh.at[pl.ds(9600, 400)],
                        pres_hbm.at[cid, pl.ds(9600, 400)])


_CH3 = 160
_NFULL3 = (_E // 32) // _CH3
_TAIL3 = (_E // 32) - _NFULL3 * _CH3


@functools.partial(
    pl.kernel, mesh=_MESH, compiler_params=_SCP,
    out_type=jax.ShapeDtypeStruct((2, _N, _C), jnp.float32),
    scratch_types=[
        pltpu.VMEM((_CH3,), jnp.int32),
        pltpu.VMEM((_CH3,), jnp.int32),
        pltpu.VMEM((_CH3,), jnp.float32),
        pltpu.VMEM((_CH3, _C), jnp.float32),
        pltpu.VMEM((_CH3,), jnp.int32),
        pltpu.VMEM((_CH3,), jnp.int32),
        pltpu.VMEM((_CH3,), jnp.float32),
        pltpu.VMEM((_CH3, _C), jnp.float32),
        pltpu.VMEM_SHARED((_N, _C), jnp.float32),
        pltpu.SemaphoreType.DMA,
        pltpu.SemaphoreType.DMA,
    ],
)
def _k3(h1_hbm, ei_hbm, ew_hbm, acc_hbm,
        srcv0, dstv0, ewv0, rowsv0, srcv1, dstv1, ewv1, rowsv1,
        acc_sh, sem0, sem1):
    cid = lax.axis_index("c")
    sid = lax.axis_index("s")
    w = _wid()
    def zr(i, c):
        for jj in range(8):
            rowsv0[i, pl.ds(jj * 16, 16)] = _ZF()
        return c
    lax.fori_loop(0, _CH3, zr, 0)
    for z in range(3):
        pltpu.sync_copy(rowsv0, acc_sh.at[pl.ds(sid * 625 + z * _CH3, _CH3)])
    pltpu.sync_copy(rowsv0.at[pl.ds(0, 625 - 3 * _CH3)],
                    acc_sh.at[pl.ds(sid * 625 + 3 * _CH3, 625 - 3 * _CH3)])
    plsc.subcore_barrier()

    base_w = w * (_E // 32)

    def load_idx(off, srcv, dstv, ewv, n=_CH3):
        pltpu.sync_copy(ei_hbm.at[0, pl.ds(off, n)], srcv.at[pl.ds(0, n)])
        pltpu.sync_copy(ei_hbm.at[1, pl.ds(off, n)], dstv.at[pl.ds(0, n)])
        pltpu.sync_copy(ew_hbm.at[pl.ds(off, n)], ewv.at[pl.ds(0, n)])

    def compute_scatter(dstv, ewv, rowsv):
        def mul(r, c):
            sc = plsc.load_gather(ewv, [_ZI() + r])
            for jj in range(8):
                rowsv[r, pl.ds(jj * 16, 16)] = (
                    rowsv[r, pl.ds(jj * 16, 16)] * sc)
            return c
        lax.fori_loop(0, _CH3, mul, 0)
        pltpu.sync_copy(rowsv, acc_sh.at[dstv], add=True)

    load_idx(base_w, srcv0, dstv0, ewv0)
    g0 = pltpu.async_copy(h1_hbm.at[srcv0], rowsv0, sem0)

    def body(jj, c):
        off1 = base_w + (2 * jj + 1) * _CH3
        load_idx(off1, srcv1, dstv1, ewv1)
        g1 = pltpu.async_copy(h1_hbm.at[srcv1], rowsv1, sem1)
        pltpu.make_async_copy(h1_hbm.at[srcv0], rowsv0, sem0).wait()
        compute_scatter(dstv0, ewv0, rowsv0)

        @pl.when(jj < _NFULL3 // 2 - 1)
        def _():
            off2 = base_w + (2 * jj + 2) * _CH3
            load_idx(off2, srcv0, dstv0, ewv0)
            pltpu.async_copy(h1_hbm.at[srcv0], rowsv0, sem0)
        g1.wait()
        compute_scatter(dstv1, ewv1, rowsv1)
        return c

    lax.fori_loop(0, _NFULL3 // 2, body, 0)

    toff = base_w + _NFULL3 * _CH3
    load_idx(toff, srcv0, dstv0, ewv0, n=_TAIL3)
    pltpu.async_copy(h1_hbm.at[srcv0.at[pl.ds(0, _TAIL3)]],
                     rowsv0.at[pl.ds(0, _TAIL3)], sem0).wait()

    def mult(r, c):
        sc = plsc.load_gather(ewv0, [_ZI() + r])
        for jj in range(8):
            rowsv0[r, pl.ds(jj * 16, 16)] = rowsv0[r, pl.ds(jj * 16, 16)] * sc
        return c
    lax.fori_loop(0, _TAIL3, mult, 0)
    pltpu.sync_copy(rowsv0.at[pl.ds(0, _TAIL3)],
                    acc_sh.at[dstv0.at[pl.ds(0, _TAIL3)]], add=True)

    plsc.subcore_barrier()
    pltpu.sync_copy(acc_sh.at[pl.ds(sid * 625, 625)],
                    acc_hbm.at[cid, pl.ds(sid * 625, 625)])


def _k2_body(acc_ref, x_ref, w1_ref, w2_ref, w3_ref, h1_ref):
    acc = acc_ref[0] + acc_ref[1]
    deg = acc[:, 4:5]
    x = x_ref[...]
    t1 = jnp.dot(acc, w1_ref[...], preferred_element_type=jnp.float32)
    t2 = jnp.dot(x, w2_ref[...], preferred_element_type=jnp.float32)
    t3 = jnp.dot(x, w3_ref[...], preferred_element_type=jnp.float32)
    h1_ref[...] = jnp.maximum(t1 - deg * t2 + t3, 0.0)


_k2 = pl.pallas_call(
    _k2_body,
    grid=(5,),
    in_specs=[
        pl.BlockSpec((2, 2000, 16), lambda i: (0, i, 0)),
        pl.BlockSpec((2000, 16), lambda i: (i, 0)),
        pl.BlockSpec((16, _C), lambda i: (0, 0)),
        pl.BlockSpec((16, _C), lambda i: (0, 0)),
        pl.BlockSpec((16, _C), lambda i: (0, 0)),
    ],
    out_specs=pl.BlockSpec((2000, _C), lambda i: (i, 0)),
    out_shape=jax.ShapeDtypeStruct((_N, _C), jnp.float32),
)


def _k4_body(acc2_ref, acc1_ref, h1_ref, w1_ref, w2_ref, w3_ref, b1_ref,
             b3_ref, h_ref):
    z2 = acc2_ref[0] + acc2_ref[1]
    deg = acc1_ref[0][:, 4:5] + acc1_ref[1][:, 4:5]
    h1 = h1_ref[...]
    t1 = jnp.dot(z2, w1_ref[...], preferred_element_type=jnp.float32)
    t2 = jnp.dot(h1, w2_ref[...], preferred_element_type=jnp.float32)
    t3 = jnp.dot(h1, w3_ref[...], preferred_element_type=jnp.float32)
    h_ref[...] = t1 + deg * b1_ref[...] - deg * t2 + t3 + b3_ref[...]


_k4 = pl.pallas_call(
    _k4_body,
    grid=(5,),
    in_specs=[
        pl.BlockSpec((2, 2000, _C), lambda i: (0, i, 0)),
        pl.BlockSpec((2, 2000, 16), lambda i: (0, i, 0)),
        pl.BlockSpec((2000, _C), lambda i: (i, 0)),
        pl.BlockSpec((_C, _C), lambda i: (0, 0)),
        pl.BlockSpec((_C, _C), lambda i: (0, 0)),
        pl.BlockSpec((_C, _C), lambda i: (0, 0)),
        pl.BlockSpec((1, _C), lambda i: (0, 0)),
        pl.BlockSpec((1, _C), lambda i: (0, 0)),
    ],
    out_specs=pl.BlockSpec((2000, _C), lambda i: (i, 0)),
    out_shape=jax.ShapeDtypeStruct((_N, _C), jnp.float32),
)


_W5 = _E // 16
_S5 = _W5 // 16
_RDX = 256
_KCH = 2000


@functools.partial(
    pl.kernel, mesh=_MESH, compiler_params=_SCP,
    out_type=(jax.ShapeDtypeStruct((_R,), jnp.float32),
              jax.ShapeDtypeStruct((_R,), jnp.float32),
              jax.ShapeDtypeStruct((_R,), jnp.int32),
              jax.ShapeDtypeStruct((_R,), jnp.int32),
              jax.ShapeDtypeStruct((_N,), jnp.float32)),
    scratch_types=[
        pltpu.VMEM((_W5,), jnp.float32),
        pltpu.VMEM((_W5,), jnp.int32),
        pltpu.VMEM((_W5,), jnp.int32),
        pltpu.VMEM((32 * _RDX,), jnp.int32),
        pltpu.VMEM((16 * _RDX,), jnp.int32),
        pltpu.VMEM((_RDX,), jnp.int32),
        pltpu.VMEM((_KCH,), jnp.float32),
        pltpu.VMEM((_KCH,), jnp.int32),
        pltpu.VMEM((_KCH,), jnp.int32),
        pltpu.VMEM((_KCH,), jnp.float32),
        pltpu.VMEM((640,), jnp.float32),
        pltpu.VMEM_SHARED((_E,), jnp.int32),
        pltpu.VMEM_SHARED((_E,), jnp.int32),
        pltpu.VMEM_SHARED((16 * _RDX,), jnp.int32),
        pltpu.VMEM_SHARED((_N,), jnp.float32),
        pltpu.SemaphoreType.DMA,
    ],
)
def _k5(score_hbm, ew_hbm, src_hbm, dst_hbm,
        rew_hbm, rea_hbm, rsrc_hbm, rdst_hbm, presr_hbm,
        keysf, valsv, posv, histv, Tallv, Tv, s2k, i2k, j2k, ones2k,
        zi640, A_v, B_v, Tall_sh, pres_r_sh, sem):
    cid = lax.axis_index("c")
    sid = lax.axis_index("s")
    lanes = _LANES()

    @pl.when(cid == 0)
    def _():
        base = sid * _W5
        _zero_vmem(zi640, 640, jnp.float32)

        @pl.when(sid < 15)
        def _():
            pltpu.sync_copy(zi640, pres_r_sh.at[pl.ds(sid * 640, 640)])

        @pl.when(sid == 15)
        def _():
            pltpu.sync_copy(zi640.at[pl.ds(0, 400)],
                            pres_r_sh.at[pl.ds(9600, 400)])

        def o1(i, c):
            ones2k[pl.ds(i * 16, 16)] = _ZF() + 1.0
            return c
        lax.fori_loop(0, _KCH // 16, o1, 0)

        for p in range(4):
            shift = 8 * p
            if p % 2 == 0:
                src_v, dst_v = A_v, B_v
            else:
                src_v, dst_v = B_v, A_v
            if p == 0:
                pltpu.sync_copy(score_hbm.at[pl.ds(base, _W5)], keysf)

                def iv(i, c):
                    valsv[pl.ds(i * 16, 16)] = base + i * 16 + lanes
                    return c
                lax.fori_loop(0, _W5 // 16, iv, 0)
            else:
                pltpu.sync_copy(src_v.at[pl.ds(base, _W5)], valsv)
                pltpu.async_copy(score_hbm.at[valsv], keysf, sem).wait()

            def mk(i, c):
                sl = pl.ds(i * 16, 16)
                keysf[sl] = plsc.bitcast(
                    _KEYMAX - plsc.bitcast(keysf[sl], jnp.int32), jnp.float32)
                return c
            lax.fori_loop(0, _W5 // 16, mk, 0)

            def zh(i, c):
                histv[pl.ds(i * 16, 16)] = _ZI()
                return c
            lax.fori_loop(0, 2 * _RDX, zh, 0)

            def ha(t, c):
                for h in range(2):
                    sl = lanes * _S5 + h * (_S5 // 2) + t
                    kv = plsc.bitcast(plsc.load_gather(keysf, [sl]),
                                      jnp.int32)
                    d = jnp.bitwise_and(lax.shift_right_logical(kv, shift),
                                        _RDX - 1)
                    fl = (lanes * 2 + h) * _RDX + d
                    cnt = plsc.load_gather(histv, [fl])
                    plsc.store_scatter(histv, [fl], cnt + 1)
                    plsc.store_scatter(posv, [sl], cnt)
                return c
            lax.fori_loop(0, _S5 // 2, ha, 0)

            def wt(j, c):
                acc = _ZI()
                for l in range(32):
                    acc = acc + histv[pl.ds(l * _RDX + j * 16, 16)]
                Tv[pl.ds(j * 16, 16)] = acc
                return c
            lax.fori_loop(0, _RDX // 16, wt, 0)
            pltpu.sync_copy(Tv, Tall_sh.at[pl.ds(sid * _RDX, _RDX)])
            plsc.subcore_barrier()

            pltpu.sync_copy(Tall_sh, Tallv)

            def pb(j, carry):
                G = _ZI()
                Wp = _ZI()
                for wk in range(16):
                    s_w = Tallv[pl.ds(wk * _RDX + j * 16, 16)]
                    G = G + s_w
                    Wp = Wp + s_w * jnp.where(wk < sid, 1, 0)
                cs = plsc.cumsum(G)
                Px = carry + cs - G
                run = Px + Wp
                for l in range(32):
                    sl_ = pl.ds(l * _RDX + j * 16, 16)
                    tmp = histv[sl_]
                    histv[sl_] = run
                    run = run + tmp
                return carry + jnp.sum(G)
            lax.fori_loop(0, _RDX // 16, pb, jnp.int32(0))

            def pc(t, c):
                for h in range(2):
                    sl = lanes * _S5 + h * (_S5 // 2) + t
                    kv = plsc.bitcast(plsc.load_gather(keysf, [sl]),
                                      jnp.int32)
                    d = jnp.bitwise_and(lax.shift_right_logical(kv, shift),
                                        _RDX - 1)
                    base_off = plsc.load_gather(
                        histv, [(lanes * 2 + h) * _RDX + d])
                    ordn = plsc.load_gather(posv, [sl])
                    plsc.store_scatter(posv, [sl], base_off + ordn)
                return c
            lax.fori_loop(0, _S5 // 2, pc, 0)

            pltpu.sync_copy(valsv, dst_v.at[posv])
            plsc.subcore_barrier()

        base2 = sid * (_R // 16)
        for q in range((_R // 16) // _KCH):
            off2 = base2 + q * _KCH
            pltpu.sync_copy(A_v.at[pl.ds(off2, _KCH)], i2k)
            pltpu.async_copy(score_hbm.at[i2k], s2k, sem).wait()
            pltpu.sync_copy(s2k, rew_hbm.at[pl.ds(off2, _KCH)])
            pltpu.async_copy(ew_hbm.at[i2k], s2k, sem).wait()
            pltpu.sync_copy(s2k, rea_hbm.at[pl.ds(off2, _KCH)])
            pltpu.async_copy(src_hbm.at[i2k], j2k, sem).wait()
            pltpu.sync_copy(j2k, rsrc_hbm.at[pl.ds(off2, _KCH)])
            pltpu.sync_copy(ones2k, pres_r_sh.at[j2k], add=True)
            pltpu.async_copy(dst_hbm.at[i2k], j2k, sem).wait()
            pltpu.sync_copy(j2k, rdst_hbm.at[pl.ds(off2, _KCH)])
            pltpu.sync_copy(ones2k, pres_r_sh.at[j2k], add=True)

        plsc.subcore_barrier()

        @pl.when(sid < 15)
        def _():
            pltpu.sync_copy(pres_r_sh.at[pl.ds(sid * 640, 640)],
                            presr_hbm.at[pl.ds(sid * 640, 640)])

        @pl.when(sid == 15)
        def _():
            pltpu.sync_copy(pres_r_sh.at[pl.ds(9600, 400)],
                            presr_hbm.at[pl.ds(9600, 400)])


def _k7_body(pf_ref, pr_ref, nif_ref, nir_ref):
    r128 = lax.broadcasted_iota(jnp.int32, (_C, _C), 0)
    c128 = lax.broadcasted_iota(jnp.int32, (_C, _C), 1)
    Lm = (r128 <= c128).astype(jnp.float32)
    Jm = jnp.ones((_C, _C), jnp.float32)
    r80 = lax.broadcasted_iota(jnp.int32, (80, 80), 0)
    c80 = lax.broadcasted_iota(jnp.int32, (80, 80), 1)
    SL = (c80 < r80).astype(jnp.float32)

    def ranks(p):
        pfl = (p > 0).astype(jnp.float32)
        incl = (jnp.dot(pfl, Lm, preferred_element_type=jnp.float32)
                + jnp.dot(SL, jnp.dot(pfl, Jm,
                                      preferred_element_type=jnp.float32),
                          preferred_element_type=jnp.float32))
        return jnp.where(pfl > 0, incl - pfl, -1.0).astype(jnp.int32)

    nif_ref[...] = ranks(pf_ref[0] + pf_ref[1])
    nir_ref[...] = ranks(pr_ref[...])


_k7 = pl.pallas_call(
    _k7_body,
    in_specs=[pl.BlockSpec((2, 80, _C), lambda: (0, 0, 0)),
              pl.BlockSpec((80, _C), lambda: (0, 0))],
    out_specs=(pl.BlockSpec((80, _C), lambda: (0, 0)),
               pl.BlockSpec((80, _C), lambda: (0, 0))),
    out_shape=(jax.ShapeDtypeStruct((80, _C), jnp.int32),
               jax.ShapeDtypeStruct((80, _C), jnp.int32)),
)


_NCH = 400
_ECH = _E // 32
_RCH = _R // 32


@functools.partial(
    pl.kernel, mesh=_MESH, compiler_params=_SCP,
    out_type=(jax.ShapeDtypeStruct((_N, _C), jnp.float32),
              jax.ShapeDtypeStruct((_N, _C), jnp.float32),
              jax.ShapeDtypeStruct((_N,), jnp.int32),
              jax.ShapeDtypeStruct((_N,), jnp.int32),
              jax.ShapeDtypeStruct((2, _R), jnp.int32),
              jax.ShapeDtypeStruct((2, _E), jnp.int32)),
    scratch_types=[
        pltpu.VMEM((_NCH,), jnp.int32),
        pltpu.VMEM((_NCH,), jnp.int32),
        pltpu.VMEM((_NCH,), jnp.int32),
        pltpu.VMEM((_NCH, _C), jnp.float32),
        pltpu.VMEM((_ECH,), jnp.int32),
        pltpu.VMEM((_ECH,), jnp.int32),
        pltpu.VMEM((_RCH,), jnp.int32),
        pltpu.VMEM((_RCH,), jnp.int32),
        pltpu.VMEM((_N + 16,), jnp.int32),
        pltpu.VMEM_SHARED((_N + 16,), jnp.int32),
        pltpu.VMEM_SHARED((_N + 16,), jnp.int32),
        pltpu.VMEM_SHARED((_N,), jnp.int32),
        pltpu.VMEM_SHARED((_N,), jnp.int32),
        pltpu.VMEM_SHARED((_N,), jnp.int32),
        pltpu.SemaphoreType.DMA,
    ],
)
def _k8(h_hbm, batch_hbm, nif_hbm, nir_hbm, rsrc_hbm, rdst_hbm, src_hbm,
        dst_hbm, rx_hbm, fx_hbm, rb_hbm, fb_hbm, rei_hbm, fei_hbm,
        nv, posv, valv, rowsv, eb, ob, eb7, ob7, big, sub_f, sub_r, nif_sh,
        nir_sh, batch_sh, sem):
    cid = lax.axis_index("c")
    sid = lax.axis_index("s")
    w = _wid()
    lanes = _LANES()
    @pl.when(sid == 0)
    def _():
        pltpu.sync_copy(nif_hbm, big.at[pl.ds(0, _N)])
        pltpu.sync_copy(big.at[pl.ds(0, _N)], nif_sh)

    @pl.when(sid == 1)
    def _():
        pltpu.sync_copy(nir_hbm, big.at[pl.ds(0, _N)])
        pltpu.sync_copy(big.at[pl.ds(0, _N)], nir_sh)

    @pl.when(sid == 2)
    def _():
        pltpu.sync_copy(batch_hbm, big.at[pl.ds(0, _N)])
        pltpu.sync_copy(big.at[pl.ds(0, _N)], batch_sh)

    @pl.when(sid == 3)
    def _():
        _zero_vmem(big, _N + 16, jnp.int32)
        pltpu.sync_copy(big, sub_f)

    @pl.when(sid == 4)
    def _():
        _zero_vmem(big, _N + 16, jnp.int32)
        pltpu.sync_copy(big, sub_r)

    plsc.subcore_barrier()

    for br in range(2):
        ni_sh = nif_sh if br == 0 else nir_sh
        sub_sh = sub_f if br == 0 else sub_r
        for c in range(_N // _NCH):
            @pl.when(c % 16 == sid)
            def _(c=c, ni_sh=ni_sh, sub_sh=sub_sh):
                pltpu.sync_copy(ni_sh.at[pl.ds(c * _NCH, _NCH)], nv)

                def bld(i, cc):
                    pos = nv[pl.ds(i * 16, 16)]
                    posv[pl.ds(i * 16, 16)] = jnp.where(
                        pos < 0, _N + lanes, pos)
                    valv[pl.ds(i * 16, 16)] = c * _NCH + i * 16 + lanes
                    return cc
                lax.fori_loop(0, _NCH // 16, bld, 0)
                pltpu.sync_copy(valv, sub_sh.at[posv])
    plsc.subcore_barrier()

    for t in range(2):
        cidx = w + 32 * t

        @pl.when(cidx < _N // _NCH)
        def _(cidx=cidx):
            off = cidx * _NCH
            pltpu.sync_copy(sub_r.at[pl.ds(off, _NCH)], nv)
            pltpu.async_copy(h_hbm.at[nv], rowsv, sem).wait()
            pltpu.sync_copy(rowsv, rx_hbm.at[pl.ds(off, _NCH)])
            pltpu.sync_copy(batch_sh.at[nv], posv)
            pltpu.sync_copy(posv, rb_hbm.at[pl.ds(off, _NCH)])

        @pl.when(jnp.logical_and(cidx >= _N // _NCH,
                                 cidx < 2 * (_N // _NCH)))
        def _(cidx=cidx):
            off = (cidx - _N // _NCH) * _NCH
            pltpu.sync_copy(sub_f.at[pl.ds(off, _NCH)], nv)
            pltpu.async_copy(h_hbm.at[nv], rowsv, sem).wait()
            pltpu.sync_copy(rowsv, fx_hbm.at[pl.ds(off, _NCH)])
            pltpu.sync_copy(batch_sh.at[nv], posv)
            pltpu.sync_copy(posv, fb_hbm.at[pl.ds(off, _NCH)])

    eoff = w * _ECH
    pltpu.sync_copy(src_hbm.at[pl.ds(eoff, _ECH)], eb)
    pltpu.sync_copy(nif_sh.at[eb], ob)
    pltpu.sync_copy(ob, fei_hbm.at[0, pl.ds(eoff, _ECH)])
    pltpu.sync_copy(dst_hbm.at[pl.ds(eoff, _ECH)], eb)
    pltpu.sync_copy(nif_sh.at[eb], ob)
    pltpu.sync_copy(ob, fei_hbm.at[1, pl.ds(eoff, _ECH)])
    roff = w * _RCH
    pltpu.sync_copy(rsrc_hbm.at[pl.ds(roff, _RCH)], eb7)
    pltpu.sync_copy(nir_sh.at[eb7], ob7)
    pltpu.sync_copy(ob7, rei_hbm.at[0, pl.ds(roff, _RCH)])
    pltpu.sync_copy(rdst_hbm.at[pl.ds(roff, _RCH)], eb7)
    pltpu.sync_copy(nir_sh.at[eb7], ob7)
    pltpu.sync_copy(ob7, rei_hbm.at[1, pl.ds(roff, _RCH)])


def kernel(x, edge_index, edge_attr, batch, edge_score, c1_w1, c1_b1, c1_w2,
           c1_w3, c1_b3, c2_w1, c2_b1, c2_w2, c2_w3, c2_b3):
    src = edge_index[0]
    dst = edge_index[1]
    ew = edge_attr.reshape(-1)
    xe = jnp.concatenate(
        [x, jnp.ones((_N, 1), jnp.float32), jnp.zeros((_N, 11), jnp.float32)],
        axis=1)

    acc1, pres_f = _k1(xe, edge_index, ew)

    zw = jnp.zeros((16, _C), jnp.float32)
    w1a = zw.at[0:4].set(c1_w1).at[4].set(c1_b1)
    w2a = zw.at[0:4].set(c1_w2)
    w3a = zw.at[0:4].set(c1_w3).at[4].set(c1_b3)
    h1 = _k2(acc1, xe, w1a, w2a, w3a)

    acc2 = _k3(h1, edge_index, ew)
    h = _k4(acc2, acc1, h1, c2_w1, c2_w2, c2_w3, c2_b1[None, :],
            c2_b3[None, :])

    r_ew, r_ea, r_src, r_dst, pres_r = _k5(edge_score, ew, src, dst)

    pf = jnp.pad(pres_f, ((0, 0), (0, 10240 - _N))).reshape(2, 80, _C)
    pr = jnp.pad(pres_r, (0, 10240 - _N)).reshape(80, _C)
    nif80, nir80 = _k7(pf, pr)
    nif = nif80.reshape(-1)[:_N]
    nir = nir80.reshape(-1)[:_N]

    r_x, f_x, r_batch, f_batch, r_ei2, f_ei2 = _k8(
        h, batch, nif, nir, r_src, r_dst, src, dst)

    return ((r_x, r_ei2, r_ea, r_ew, r_batch),
            (f_x, f_ei2, ew, edge_score, f_batch),
            edge_score)


# speedup vs baseline: 18.1995x; 1.0101x over previous
import functools
import jax
import jax.numpy as jnp
from jax import lax
from jax.experimental import pallas as pl
from jax.experimental.pallas import tpu as pltpu, tpu_sc as plsc

_N = 10000
_E = 320000
_C = 128
_R = 224000
_KEYMAX = 0x3F7FFFFF

_MESH = plsc.VectorSubcoreMesh(core_axis_name="c", subcore_axis_name="s")
_SCP = pltpu.CompilerParams(needs_layout_passes=False, use_tc_tiling_on_sc=False)

_LANES = lambda: lax.iota(jnp.int32, 16)
_ZI = lambda: jnp.zeros((16,), jnp.int32)
_ZF = lambda: jnp.zeros((16,), jnp.float32)


def _wid():
    return lax.axis_index("c") * 16 + lax.axis_index("s")


def _zero_vmem(ref, n, dtype):
    z = jnp.zeros((16,), dtype)

    def body(i, c):
        ref[pl.ds(i * 16, 16)] = z
        return c

    lax.fori_loop(0, n // 16, body, 0)


_CH1 = 1000


@functools.partial(
    pl.kernel, mesh=_MESH, compiler_params=_SCP,
    out_type=(jax.ShapeDtypeStruct((2, _N, 16), jnp.float32),
              jax.ShapeDtypeStruct((2, _N), jnp.float32)),
    scratch_types=[
        pltpu.VMEM((_CH1,), jnp.int32),
        pltpu.VMEM((_CH1,), jnp.int32),
        pltpu.VMEM((_CH1,), jnp.float32),
        pltpu.VMEM((_CH1, 16), jnp.float32),
        pltpu.VMEM((_CH1,), jnp.int32),
        pltpu.VMEM((_CH1,), jnp.int32),
        pltpu.VMEM((_CH1,), jnp.float32),
        pltpu.VMEM((_CH1, 16), jnp.float32),
        pltpu.VMEM((_CH1,), jnp.float32),
        pltpu.VMEM((640, 16), jnp.float32),
        pltpu.VMEM((640,), jnp.float32),
        pltpu.VMEM_SHARED((_N, 16), jnp.float32),
        pltpu.VMEM_SHARED((_N,), jnp.float32),
        pltpu.SemaphoreType.DMA,
        pltpu.SemaphoreType.DMA,
    ],
)
def _k1(xe_hbm, ei_hbm, ew_hbm, acc_hbm, pres_hbm,
        srcv0, dstv0, ewv0, rowsv0, srcv1, dstv1, ewv1, rowsv1,
        onesv, zrows, zi640, acc_sh, pres_sh, sem0, sem1):
    cid = lax.axis_index("c")
    sid = lax.axis_index("s")
    w = _wid()
    lanes = _LANES()
    def zr(i, c):
        zrows[i, :] = _ZF()
        return c
    lax.fori_loop(0, 640, zr, 0)
    _zero_vmem(zi640, 640, jnp.float32)
    pltpu.sync_copy(zrows.at[pl.ds(0, 625)], acc_sh.at[pl.ds(sid * 625, 625)])

    @pl.when(sid < 15)
    def _():
        pltpu.sync_copy(zi640, pres_sh.at[pl.ds(sid * 640, 640)])

    @pl.when(sid == 15)
    def _():
        pltpu.sync_copy(zi640.at[pl.ds(0, 400)], pres_sh.at[pl.ds(9600, 400)])

    def o1(i, c):
        onesv[pl.ds(i * 16, 16)] = _ZF() + 1.0
        return c
    lax.fori_loop(0, _CH1 // 16, o1, 0)
    plsc.subcore_barrier()

    base_w = w * (_E // 32)
    _NC1 = (_E // 32) // _CH1

    def load_idx(off, srcv, dstv, ewv):
        pltpu.sync_copy(ei_hbm.at[0, pl.ds(off, _CH1)], srcv)
        pltpu.sync_copy(ei_hbm.at[1, pl.ds(off, _CH1)], dstv)
        pltpu.sync_copy(ew_hbm.at[pl.ds(off, _CH1)], ewv)

    def compute_scatter(srcv, dstv, ewv, rowsv):
        def mul(r, c):
            sc = plsc.load_gather(ewv, [_ZI() + r])
            rowsv[r, :] = rowsv[r, :] * sc
            return c
        lax.fori_loop(0, _CH1, mul, 0)
        pltpu.sync_copy(rowsv, acc_sh.at[dstv], add=True)
        pltpu.sync_copy(onesv, pres_sh.at[srcv], add=True)
        pltpu.sync_copy(onesv, pres_sh.at[dstv], add=True)

    load_idx(base_w, srcv0, dstv0, ewv0)
    pltpu.async_copy(xe_hbm.at[srcv0], rowsv0, sem0)

    def body(jj, c):
        off1 = base_w + (2 * jj + 1) * _CH1
        load_idx(off1, srcv1, dstv1, ewv1)
        g1 = pltpu.async_copy(xe_hbm.at[srcv1], rowsv1, sem1)
        pltpu.make_async_copy(xe_hbm.at[srcv0], rowsv0, sem0).wait()
        compute_scatter(srcv0, dstv0, ewv0, rowsv0)

        @pl.when(jj < _NC1 // 2 - 1)
        def _():
            off2 = base_w + (2 * jj + 2) * _CH1
            load_idx(off2, srcv0, dstv0, ewv0)
            pltpu.async_copy(xe_hbm.at[srcv0], rowsv0, sem0)
        g1.wait()
        compute_scatter(srcv1, dstv1, ewv1, rowsv1)
        return c

    lax.fori_loop(0, _NC1 // 2, body, 0)

    plsc.subcore_barrier()
    pltpu.sync_copy(acc_sh.at[pl.ds(sid * 625, 625)],
                    acc_hbm.at[cid, pl.ds(sid * 625, 625)])

    @pl.when(sid < 15)
    def _():
        pltpu.sync_copy(pres_sh.at[pl.ds(sid * 640, 640)],
                        pres_hbm.at[cid, pl.ds(sid * 640, 640)])

    @pl.when(sid == 15)
    def _():
        pltpu.sync_copy(pres_sh.at[pl.ds(9600, 400)],
                        pres_hbm.at[cid, pl.ds(9600, 400)])


_CH3 = 160
_NFULL3 = (_E // 32) // _CH3
_TAIL3 = (_E // 32) - _NFULL3 * _CH3


@functools.partial(
    pl.kernel, mesh=_MESH, compiler_params=_SCP,
    out_type=jax.ShapeDtypeStruct((2, _N, _C), jnp.float32),
    scratch_types=[
        pltpu.VMEM((_CH3,), jnp.int32),
        pltpu.VMEM((_CH3,), jnp.int32),
        pltpu.VMEM((_CH3,), jnp.float32),
        pltpu.VMEM((_CH3, _C), jnp.float32),
        pltpu.VMEM((_CH3,), jnp.int32),
        pltpu.VMEM((_CH3,), jnp.int32),
        pltpu.VMEM((_CH3,), jnp.float32),
        pltpu.VMEM((_CH3, _C), jnp.float32),
        pltpu.VMEM_SHARED((_N, _C), jnp.float32),
        pltpu.SemaphoreType.DMA,
        pltpu.SemaphoreType.DMA,
    ],
)
def _k3(h1_hbm, ei_hbm, ew_hbm, acc_hbm,
        srcv0, dstv0, ewv0, rowsv0, srcv1, dstv1, ewv1, rowsv1,
        acc_sh, sem0, sem1):
    cid = lax.axis_index("c")
    sid = lax.axis_index("s")
    w = _wid()
    def zr(i, c):
        for jj in range(8):
            rowsv0[i, pl.ds(jj * 16, 16)] = _ZF()
        return c
    lax.fori_loop(0, _CH3, zr, 0)
    for z in range(3):
        pltpu.sync_copy(rowsv0, acc_sh.at[pl.ds(sid * 625 + z * _CH3, _CH3)])
    pltpu.sync_copy(rowsv0.at[pl.ds(0, 625 - 3 * _CH3)],
                    acc_sh.at[pl.ds(sid * 625 + 3 * _CH3, 625 - 3 * _CH3)])
    plsc.subcore_barrier()

    base_w = w * (_E // 32)

    def load_idx(off, srcv, dstv, ewv, n=_CH3):
        pltpu.sync_copy(ei_hbm.at[0, pl.ds(off, n)], srcv.at[pl.ds(0, n)])
        pltpu.sync_copy(ei_hbm.at[1, pl.ds(off, n)], dstv.at[pl.ds(0, n)])
        pltpu.sync_copy(ew_hbm.at[pl.ds(off, n)], ewv.at[pl.ds(0, n)])

    def compute_scatter(dstv, ewv, rowsv):
        def mul(r, c):
            sc = plsc.load_gather(ewv, [_ZI() + r])
            for jj in range(8):
                rowsv[r, pl.ds(jj * 16, 16)] = (
                    rowsv[r, pl.ds(jj * 16, 16)] * sc)
            return c
        lax.fori_loop(0, _CH3, mul, 0)
        pltpu.sync_copy(rowsv, acc_sh.at[dstv], add=True)

    load_idx(base_w, srcv0, dstv0, ewv0)
    g0 = pltpu.async_copy(h1_hbm.at[srcv0], rowsv0, sem0)

    def body(jj, c):
        off1 = base_w + (2 * jj + 1) * _CH3
        load_idx(off1, srcv1, dstv1, ewv1)
        g1 = pltpu.async_copy(h1_hbm.at[srcv1], rowsv1, sem1)
        pltpu.make_async_copy(h1_hbm.at[srcv0], rowsv0, sem0).wait()
        compute_scatter(dstv0, ewv0, rowsv0)

        @pl.when(jj < _NFULL3 // 2 - 1)
        def _():
            off2 = base_w + (2 * jj + 2) * _CH3
            load_idx(off2, srcv0, dstv0, ewv0)
            pltpu.async_copy(h1_hbm.at[srcv0], rowsv0, sem0)
        g1.wait()
        compute_scatter(dstv1, ewv1, rowsv1)
        return c

    lax.fori_loop(0, _NFULL3 // 2, body, 0)

    toff = base_w + _NFULL3 * _CH3
    load_idx(toff, srcv0, dstv0, ewv0, n=_TAIL3)
    pltpu.async_copy(h1_hbm.at[srcv0.at[pl.ds(0, _TAIL3)]],
                     rowsv0.at[pl.ds(0, _TAIL3)], sem0).wait()

    def mult(r, c):
        sc = plsc.load_gather(ewv0, [_ZI() + r])
        for jj in range(8):
            rowsv0[r, pl.ds(jj * 16, 16)] = rowsv0[r, pl.ds(jj * 16, 16)] * sc
        return c
    lax.fori_loop(0, _TAIL3, mult, 0)
    pltpu.sync_copy(rowsv0.at[pl.ds(0, _TAIL3)],
                    acc_sh.at[dstv0.at[pl.ds(0, _TAIL3)]], add=True)

    plsc.subcore_barrier()
    pltpu.sync_copy(acc_sh.at[pl.ds(sid * 625, 625)],
                    acc_hbm.at[cid, pl.ds(sid * 625, 625)])


def _k2_body(acc_ref, x_ref, w1_ref, w2_ref, w3_ref, h1_ref):
    acc = acc_ref[0] + acc_ref[1]
    deg = acc[:, 4:5]
    x = x_ref[...]
    t1 = jnp.dot(acc, w1_ref[...], preferred_element_type=jnp.float32)
    t2 = jnp.dot(x, w2_ref[...], preferred_element_type=jnp.float32)
    t3 = jnp.dot(x, w3_ref[...], preferred_element_type=jnp.float32)
    h1_ref[...] = jnp.maximum(t1 - deg * t2 + t3, 0.0)


_k2 = pl.pallas_call(
    _k2_body,
    grid=(5,),
    in_specs=[
        pl.BlockSpec((2, 2000, 16), lambda i: (0, i, 0)),
        pl.BlockSpec((2000, 16), lambda i: (i, 0)),
        pl.BlockSpec((16, _C), lambda i: (0, 0)),
        pl.BlockSpec((16, _C), lambda i: (0, 0)),
        pl.BlockSpec((16, _C), lambda i: (0, 0)),
    ],
    out_specs=pl.BlockSpec((2000, _C), lambda i: (i, 0)),
    out_shape=jax.ShapeDtypeStruct((_N, _C), jnp.float32),
)


def _k4_body(acc2_ref, acc1_ref, h1_ref, w1_ref, w2_ref, w3_ref, b1_ref,
             b3_ref, h_ref):
    z2 = acc2_ref[0] + acc2_ref[1]
    deg = acc1_ref[0][:, 4:5] + acc1_ref[1][:, 4:5]
    h1 = h1_ref[...]
    t1 = jnp.dot(z2, w1_ref[...], preferred_element_type=jnp.float32)
    t2 = jnp.dot(h1, w2_ref[...], preferred_element_type=jnp.float32)
    t3 = jnp.dot(h1, w3_ref[...], preferred_element_type=jnp.float32)
    h_ref[...] = t1 + deg * b1_ref[...] - deg * t2 + t3 + b3_ref[...]


_k4 = pl.pallas_call(
    _k4_body,
    grid=(5,),
    in_specs=[
        pl.BlockSpec((2, 2000, _C), lambda i: (0, i, 0)),
        pl.BlockSpec((2, 2000, 16), lambda i: (0, i, 0)),
        pl.BlockSpec((2000, _C), lambda i: (i, 0)),
        pl.BlockSpec((_C, _C), lambda i: (0, 0)),
        pl.BlockSpec((_C, _C), lambda i: (0, 0)),
        pl.BlockSpec((_C, _C), lambda i: (0, 0)),
        pl.BlockSpec((1, _C), lambda i: (0, 0)),
        pl.BlockSpec((1, _C), lambda i: (0, 0)),
    ],
    out_specs=pl.BlockSpec((2000, _C), lambda i: (i, 0)),
    out_shape=jax.ShapeDtypeStruct((_N, _C), jnp.float32),
)


_W5 = _E // 16
_S5 = _W5 // 16
_RDX = 256
_KCH = 2000


@functools.partial(
    pl.kernel, mesh=_MESH, compiler_params=_SCP,
    out_type=(jax.ShapeDtypeStruct((_R,), jnp.float32),
              jax.ShapeDtypeStruct((_R,), jnp.float32),
              jax.ShapeDtypeStruct((_R,), jnp.int32),
              jax.ShapeDtypeStruct((_R,), jnp.int32),
              jax.ShapeDtypeStruct((_N,), jnp.float32)),
    scratch_types=[
        pltpu.VMEM((_W5,), jnp.float32),
        pltpu.VMEM((_W5,), jnp.int32),
        pltpu.VMEM((_W5,), jnp.int32),
        pltpu.VMEM((16 * _RDX,), jnp.int32),
        pltpu.VMEM((16 * _RDX,), jnp.int32),
        pltpu.VMEM((_RDX,), jnp.int32),
        pltpu.VMEM((_KCH,), jnp.float32),
        pltpu.VMEM((_KCH,), jnp.int32),
        pltpu.VMEM((_KCH,), jnp.int32),
        pltpu.VMEM((_KCH,), jnp.float32),
        pltpu.VMEM((640,), jnp.float32),
        pltpu.VMEM_SHARED((_E,), jnp.int32),
        pltpu.VMEM_SHARED((_E,), jnp.int32),
        pltpu.VMEM_SHARED((16 * _RDX,), jnp.int32),
        pltpu.VMEM_SHARED((_N,), jnp.float32),
        pltpu.SemaphoreType.DMA,
    ],
)
def _k5(score_hbm, ew_hbm, src_hbm, dst_hbm,
        rew_hbm, rea_hbm, rsrc_hbm, rdst_hbm, presr_hbm,
        keysf, valsv, posv, histv, Tallv, Tv, s2k, i2k, j2k, ones2k,
        zi640, A_v, B_v, Tall_sh, pres_r_sh, sem):
    cid = lax.axis_index("c")
    sid = lax.axis_index("s")
    lanes = _LANES()

    @pl.when(cid == 0)
    def _():
        base = sid * _W5
        _zero_vmem(zi640, 640, jnp.float32)

        @pl.when(sid < 15)
        def _():
            pltpu.sync_copy(zi640, pres_r_sh.at[pl.ds(sid * 640, 640)])

        @pl.when(sid == 15)
        def _():
            pltpu.sync_copy(zi640.at[pl.ds(0, 400)],
                            pres_r_sh.at[pl.ds(9600, 400)])

        def o1(i, c):
            ones2k[pl.ds(i * 16, 16)] = _ZF() + 1.0
            return c
        lax.fori_loop(0, _KCH // 16, o1, 0)

        for p in range(4):
            shift = 8 * p
            if p % 2 == 0:
                src_v, dst_v = A_v, B_v
            else:
                src_v, dst_v = B_v, A_v
            if p == 0:
                def iv(i, c):
                    valsv[pl.ds(i * 16, 16)] = base + lanes * _S5 + i
                    return c
                lax.fori_loop(0, _S5, iv, 0)
            else:
                pltpu.sync_copy(src_v.at[pl.ds(base, _W5)], posv)

                def tv(i, c):
                    valsv[pl.ds(i * 16, 16)] = plsc.load_gather(
                        posv, [lanes * _S5 + i])
                    return c
                lax.fori_loop(0, _S5, tv, 0)
            pltpu.async_copy(score_hbm.at[valsv], keysf, sem).wait()

            def mk(i, c):
                sl = pl.ds(i * 16, 16)
                keysf[sl] = plsc.bitcast(
                    _KEYMAX - plsc.bitcast(keysf[sl], jnp.int32), jnp.float32)
                return c
            lax.fori_loop(0, _S5, mk, 0)

            def zh(i, c):
                histv[pl.ds(i * 16, 16)] = _ZI()
                return c
            lax.fori_loop(0, _RDX, zh, 0)

            def ha(i, c):
                sl = pl.ds(i * 16, 16)
                kv = plsc.bitcast(keysf[sl], jnp.int32)
                d = jnp.bitwise_and(lax.shift_right_logical(kv, shift),
                                    _RDX - 1)
                fl = lanes * _RDX + d
                cnt = plsc.load_gather(histv, [fl])
                plsc.store_scatter(histv, [fl], cnt + 1)
                posv[sl] = cnt
                return c
            lax.fori_loop(0, _S5, ha, 0)

            def wt(j, c):
                acc = _ZI()
                for l in range(16):
                    acc = acc + histv[pl.ds(l * _RDX + j * 16, 16)]
                Tv[pl.ds(j * 16, 16)] = acc
                return c
            lax.fori_loop(0, _RDX // 16, wt, 0)
            pltpu.sync_copy(Tv, Tall_sh.at[pl.ds(sid * _RDX, _RDX)])
            plsc.subcore_barrier()

            pltpu.sync_copy(Tall_sh, Tallv)

            def pb(j, carry):
                G = _ZI()
                Wp = _ZI()
                for wk in range(16):
                    s_w = Tallv[pl.ds(wk * _RDX + j * 16, 16)]
                    G = G + s_w
                    Wp = Wp + s_w * jnp.where(wk < sid, 1, 0)
                cs = plsc.cumsum(G)
                Px = carry + cs - G
                run = Px + Wp
                for l in range(16):
                    sl_ = pl.ds(l * _RDX + j * 16, 16)
                    tmp = histv[sl_]
                    histv[sl_] = run
                    run = run + tmp
                return carry + jnp.sum(G)
            lax.fori_loop(0, _RDX // 16, pb, jnp.int32(0))

            def pc(i, c):
                sl = pl.ds(i * 16, 16)
                kv = plsc.bitcast(keysf[sl], jnp.int32)
                d = jnp.bitwise_and(lax.shift_right_logical(kv, shift),
                                    _RDX - 1)
                base_off = plsc.load_gather(histv, [lanes * _RDX + d])
                posv[sl] = posv[sl] + base_off
                return c
            lax.fori_loop(0, _S5, pc, 0)

            pltpu.sync_copy(valsv, dst_v.at[posv])
            plsc.subcore_barrier()

        base2 = sid * (_R // 16)
        for q in range((_R // 16) // _KCH):
            off2 = base2 + q * _KCH
            pltpu.sync_copy(A_v.at[pl.ds(off2, _KCH)], i2k)
            pltpu.async_copy(score_hbm.at[i2k], s2k, sem).wait()
            pltpu.sync_copy(s2k, rew_hbm.at[pl.ds(off2, _KCH)])
            pltpu.async_copy(ew_hbm.at[i2k], s2k, sem).wait()
            pltpu.sync_copy(s2k, rea_hbm.at[pl.ds(off2, _KCH)])
            pltpu.async_copy(src_hbm.at[i2k], j2k, sem).wait()
            pltpu.sync_copy(j2k, rsrc_hbm.at[pl.ds(off2, _KCH)])
            pltpu.sync_copy(ones2k, pres_r_sh.at[j2k], add=True)
            pltpu.async_copy(dst_hbm.at[i2k], j2k, sem).wait()
            pltpu.sync_copy(j2k, rdst_hbm.at[pl.ds(off2, _KCH)])
            pltpu.sync_copy(ones2k, pres_r_sh.at[j2k], add=True)

        plsc.subcore_barrier()

        @pl.when(sid < 15)
        def _():
            pltpu.sync_copy(pres_r_sh.at[pl.ds(sid * 640, 640)],
                            presr_hbm.at[pl.ds(sid * 640, 640)])

        @pl.when(sid == 15)
        def _():
            pltpu.sync_copy(pres_r_sh.at[pl.ds(9600, 400)],
                            presr_hbm.at[pl.ds(9600, 400)])


def _k7_body(pf_ref, pr_ref, nif_ref, nir_ref):
    r128 = lax.broadcasted_iota(jnp.int32, (_C, _C), 0)
    c128 = lax.broadcasted_iota(jnp.int32, (_C, _C), 1)
    Lm = (r128 <= c128).astype(jnp.float32)
    Jm = jnp.ones((_C, _C), jnp.float32)
    r80 = lax.broadcasted_iota(jnp.int32, (80, 80), 0)
    c80 = lax.broadcasted_iota(jnp.int32, (80, 80), 1)
    SL = (c80 < r80).astype(jnp.float32)

    def ranks(p):
        pfl = (p > 0).astype(jnp.float32)
        incl = (jnp.dot(pfl, Lm, preferred_element_type=jnp.float32)
                + jnp.dot(SL, jnp.dot(pfl, Jm,
                                      preferred_element_type=jnp.float32),
                          preferred_element_type=jnp.float32))
        return jnp.where(pfl > 0, incl - pfl, -1.0).astype(jnp.int32)

    nif_ref[...] = ranks(pf_ref[0] + pf_ref[1])
    nir_ref[...] = ranks(pr_ref[...])


_k7 = pl.pallas_call(
    _k7_body,
    in_specs=[pl.BlockSpec((2, 80, _C), lambda: (0, 0, 0)),
              pl.BlockSpec((80, _C), lambda: (0, 0))],
    out_specs=(pl.BlockSpec((80, _C), lambda: (0, 0)),
               pl.BlockSpec((80, _C), lambda: (0, 0))),
    out_shape=(jax.ShapeDtypeStruct((80, _C), jnp.int32),
               jax.ShapeDtypeStruct((80, _C), jnp.int32)),
)


_NCH = 400
_ECH = _E // 32
_RCH = _R // 32


@functools.partial(
    pl.kernel, mesh=_MESH, compiler_params=_SCP,
    out_type=(jax.ShapeDtypeStruct((_N, _C), jnp.float32),
              jax.ShapeDtypeStruct((_N, _C), jnp.float32),
              jax.ShapeDtypeStruct((_N,), jnp.int32),
              jax.ShapeDtypeStruct((_N,), jnp.int32),
              jax.ShapeDtypeStruct((2, _R), jnp.int32),
              jax.ShapeDtypeStruct((2, _E), jnp.int32)),
    scratch_types=[
        pltpu.VMEM((_NCH,), jnp.int32),
        pltpu.VMEM((_NCH,), jnp.int32),
        pltpu.VMEM((_NCH,), jnp.int32),
        pltpu.VMEM((_NCH, _C), jnp.float32),
        pltpu.VMEM((_ECH,), jnp.int32),
        pltpu.VMEM((_ECH,), jnp.int32),
        pltpu.VMEM((_RCH,), jnp.int32),
        pltpu.VMEM((_RCH,), jnp.int32),
        pltpu.VMEM((_N + 16,), jnp.int32),
        pltpu.VMEM_SHARED((_N + 16,), jnp.int32),
        pltpu.VMEM_SHARED((_N + 16,), jnp.int32),
        pltpu.VMEM_SHARED((_N,), jnp.int32),
        pltpu.VMEM_SHARED((_N,), jnp.int32),
        pltpu.VMEM_SHARED((_N,), jnp.int32),
        pltpu.SemaphoreType.DMA,
    ],
)
def _k8(h_hbm, batch_hbm, nif_hbm, nir_hbm, rsrc_hbm, rdst_hbm, src_hbm,
        dst_hbm, rx_hbm, fx_hbm, rb_hbm, fb_hbm, rei_hbm, fei_hbm,
        nv, posv, valv, rowsv, eb, ob, eb7, ob7, big, sub_f, sub_r, nif_sh,
        nir_sh, batch_sh, sem):
    cid = lax.axis_index("c")
    sid = lax.axis_index("s")
    w = _wid()
    lanes = _LANES()
    @pl.when(sid == 0)
    def _():
        pltpu.sync_copy(nif_hbm, big.at[pl.ds(0, _N)])
        pltpu.sync_copy(big.at[pl.ds(0, _N)], nif_sh)

    @pl.when(sid == 1)
    def _():
        pltpu.sync_copy(nir_hbm, big.at[pl.ds(0, _N)])
        pltpu.sync_copy(big.at[pl.ds(0, _N)], nir_sh)

    @pl.when(sid == 2)
    def _():
        pltpu.sync_copy(batch_hbm, big.at[pl.ds(0, _N)])
        pltpu.sync_copy(big.at[pl.ds(0, _N)], batch_sh)

    @pl.when(sid == 3)
    def _():
        _zero_vmem(big, _N + 16, jnp.int32)
        pltpu.sync_copy(big, sub_f)

    @pl.when(sid == 4)
    def _():
        _zero_vmem(big, _N + 16, jnp.int32)
        pltpu.sync_copy(big, sub_r)

    plsc.subcore_barrier()

    for br in range(2):
        ni_sh = nif_sh if br == 0 else nir_sh
        sub_sh = sub_f if br == 0 else sub_r
        for c in range(_N // _NCH):
            @pl.when(c % 16 == sid)
            def _(c=c, ni_sh=ni_sh, sub_sh=sub_sh):
                pltpu.sync_copy(ni_sh.at[pl.ds(c * _NCH, _NCH)], nv)

                def bld(i, cc):
                    pos = nv[pl.ds(i * 16, 16)]
                    posv[pl.ds(i * 16, 16)] = jnp.where(
                        pos < 0, _N + lanes, pos)
                    valv[pl.ds(i * 16, 16)] = c * _NCH + i * 16 + lanes
                    return cc
                lax.fori_loop(0, _NCH // 16, bld, 0)
                pltpu.sync_copy(valv, sub_sh.at[posv])
    plsc.subcore_barrier()

    for t in range(2):
        cidx = w + 32 * t

        @pl.when(cidx < _N // _NCH)
        def _(cidx=cidx):
            off = cidx * _NCH
            pltpu.sync_copy(sub_r.at[pl.ds(off, _NCH)], nv)
            pltpu.async_copy(h_hbm.at[nv], rowsv, sem).wait()
            pltpu.sync_copy(rowsv, rx_hbm.at[pl.ds(off, _NCH)])
            pltpu.sync_copy(batch_sh.at[nv], posv)
            pltpu.sync_copy(posv, rb_hbm.at[pl.ds(off, _NCH)])

        @pl.when(jnp.logical_and(cidx >= _N // _NCH,
                                 cidx < 2 * (_N // _NCH)))
        def _(cidx=cidx):
            off = (cidx - _N // _NCH) * _NCH
            pltpu.sync_copy(sub_f.at[pl.ds(off, _NCH)], nv)
            pltpu.async_copy(h_hbm.at[nv], rowsv, sem).wait()
            pltpu.sync_copy(rowsv, fx_hbm.at[pl.ds(off, _NCH)])
            pltpu.sync_copy(batch_sh.at[nv], posv)
            pltpu.sync_copy(posv, fb_hbm.at[pl.ds(off, _NCH)])

    eoff = w * _ECH
    pltpu.sync_copy(src_hbm.at[pl.ds(eoff, _ECH)], eb)
    pltpu.sync_copy(nif_sh.at[eb], ob)
    pltpu.sync_copy(ob, fei_hbm.at[0, pl.ds(eoff, _ECH)])
    pltpu.sync_copy(dst_hbm.at[pl.ds(eoff, _ECH)], eb)
    pltpu.sync_copy(nif_sh.at[eb], ob)
    pltpu.sync_copy(ob, fei_hbm.at[1, pl.ds(eoff, _ECH)])
    roff = w * _RCH
    pltpu.sync_copy(rsrc_hbm.at[pl.ds(roff, _RCH)], eb7)
    pltpu.sync_copy(nir_sh.at[eb7], ob7)
    pltpu.sync_copy(ob7, rei_hbm.at[0, pl.ds(roff, _RCH)])
    pltpu.sync_copy(rdst_hbm.at[pl.ds(roff, _RCH)], eb7)
    pltpu.sync_copy(nir_sh.at[eb7], ob7)
    pltpu.sync_copy(ob7, rei_hbm.at[1, pl.ds(roff, _RCH)])


def kernel(x, edge_index, edge_attr, batch, edge_score, c1_w1, c1_b1, c1_w2,
           c1_w3, c1_b3, c2_w1, c2_b1, c2_w2, c2_w3, c2_b3):
    src = edge_index[0]
    dst = edge_index[1]
    ew = edge_attr.reshape(-1)
    xe = jnp.concatenate(
        [x, jnp.ones((_N, 1), jnp.float32), jnp.zeros((_N, 11), jnp.float32)],
        axis=1)

    acc1, pres_f = _k1(xe, edge_index, ew)

    zw = jnp.zeros((16, _C), jnp.float32)
    w1a = zw.at[0:4].set(c1_w1).at[4].set(c1_b1)
    w2a = zw.at[0:4].set(c1_w2)
    w3a = zw.at[0:4].set(c1_w3).at[4].set(c1_b3)
    h1 = _k2(acc1, xe, w1a, w2a, w3a)

    acc2 = _k3(h1, edge_index, ew)
    h = _k4(acc2, acc1, h1, c2_w1, c2_w2, c2_w3, c2_b1[None, :],
            c2_b3[None, :])

    r_ew, r_ea, r_src, r_dst, pres_r = _k5(edge_score, ew, src, dst)

    pf = jnp.pad(pres_f, ((0, 0), (0, 10240 - _N))).reshape(2, 80, _C)
    pr = jnp.pad(pres_r, (0, 10240 - _N)).reshape(80, _C)
    nif80, nir80 = _k7(pf, pr)
    nif = nif80.reshape(-1)[:_N]
    nir = nir80.reshape(-1)[:_N]

    r_x, f_x, r_batch, f_batch, r_ei2, f_ei2 = _k8(
        h, batch, nif, nir, r_src, r_dst, src, dst)

    return ((r_x, r_ei2, r_ea, r_ew, r_batch),
            (f_x, f_ei2, ew, edge_score, f_batch),
            edge_score)


# speedup vs baseline: 18.6867x; 1.0268x over previous
import functools
import jax
import jax.numpy as jnp
from jax import lax
from jax.experimental import pallas as pl
from jax.experimental.pallas import tpu as pltpu, tpu_sc as plsc

_N = 10000
_E = 320000
_C = 128
_R = 224000
_KEYMAX = 0x3F7FFFFF

_MESH = plsc.VectorSubcoreMesh(core_axis_name="c", subcore_axis_name="s")
_SCP = pltpu.CompilerParams(needs_layout_passes=False, use_tc_tiling_on_sc=False)

_LANES = lambda: lax.iota(jnp.int32, 16)
_ZI = lambda: jnp.zeros((16,), jnp.int32)
_ZF = lambda: jnp.zeros((16,), jnp.float32)


def _wid():
    return lax.axis_index("c") * 16 + lax.axis_index("s")


def _zero_vmem(ref, n, dtype):
    z = jnp.zeros((16,), dtype)

    def body(i, c):
        ref[pl.ds(i * 16, 16)] = z
        return c

    lax.fori_loop(0, n // 16, body, 0)


_CH1 = 1000


@functools.partial(
    pl.kernel, mesh=_MESH, compiler_params=_SCP,
    out_type=(jax.ShapeDtypeStruct((2, _N, 16), jnp.float32),
              jax.ShapeDtypeStruct((2, _N), jnp.float32)),
    scratch_types=[
        pltpu.VMEM((_CH1,), jnp.int32),
        pltpu.VMEM((_CH1,), jnp.int32),
        pltpu.VMEM((_CH1,), jnp.float32),
        pltpu.VMEM((_CH1, 16), jnp.float32),
        pltpu.VMEM((_CH1,), jnp.int32),
        pltpu.VMEM((_CH1,), jnp.int32),
        pltpu.VMEM((_CH1,), jnp.float32),
        pltpu.VMEM((_CH1, 16), jnp.float32),
        pltpu.VMEM((_CH1,), jnp.float32),
        pltpu.VMEM((640, 16), jnp.float32),
        pltpu.VMEM((640,), jnp.float32),
        pltpu.VMEM_SHARED((_N, 16), jnp.float32),
        pltpu.VMEM_SHARED((_N,), jnp.float32),
        pltpu.SemaphoreType.DMA,
        pltpu.SemaphoreType.DMA,
    ],
)
def _k1(xe_hbm, ei_hbm, ew_hbm, acc_hbm, pres_hbm,
        srcv0, dstv0, ewv0, rowsv0, srcv1, dstv1, ewv1, rowsv1,
        onesv, zrows, zi640, acc_sh, pres_sh, sem0, sem1):
    cid = lax.axis_index("c")
    sid = lax.axis_index("s")
    w = _wid()
    lanes = _LANES()
    def zr(i, c):
        zrows[i, :] = _ZF()
        return c
    lax.fori_loop(0, 640, zr, 0)
    _zero_vmem(zi640, 640, jnp.float32)
    pltpu.sync_copy(zrows.at[pl.ds(0, 625)], acc_sh.at[pl.ds(sid * 625, 625)])

    @pl.when(sid < 15)
    def _():
        pltpu.sync_copy(zi640, pres_sh.at[pl.ds(sid * 640, 640)])

    @pl.when(sid == 15)
    def _():
        pltpu.sync_copy(zi640.at[pl.ds(0, 400)], pres_sh.at[pl.ds(9600, 400)])

    def o1(i, c):
        onesv[pl.ds(i * 16, 16)] = _ZF() + 1.0
        return c
    lax.fori_loop(0, _CH1 // 16, o1, 0)
    plsc.subcore_barrier()

    base_w = w * (_E // 32)
    _NC1 = (_E // 32) // _CH1

    def load_idx(off, srcv, dstv, ewv):
        pltpu.sync_copy(ei_hbm.at[0, pl.ds(off, _CH1)], srcv)
        pltpu.sync_copy(ei_hbm.at[1, pl.ds(off, _CH1)], dstv)
        pltpu.sync_copy(ew_hbm.at[pl.ds(off, _CH1)], ewv)

    def compute_scatter(srcv, dstv, ewv, rowsv):
        def mul(r, c):
            sc = plsc.load_gather(ewv, [_ZI() + r])
            rowsv[r, :] = rowsv[r, :] * sc
            return c
        lax.fori_loop(0, _CH1, mul, 0)
        pltpu.sync_copy(rowsv, acc_sh.at[dstv], add=True)
        pltpu.sync_copy(onesv, pres_sh.at[srcv], add=True)
        pltpu.sync_copy(onesv, pres_sh.at[dstv], add=True)

    load_idx(base_w, srcv0, dstv0, ewv0)
    pltpu.async_copy(xe_hbm.at[srcv0], rowsv0, sem0)

    def body(jj, c):
        off1 = base_w + (2 * jj + 1) * _CH1
        load_idx(off1, srcv1, dstv1, ewv1)
        g1 = pltpu.async_copy(xe_hbm.at[srcv1], rowsv1, sem1)
        pltpu.make_async_copy(xe_hbm.at[srcv0], rowsv0, sem0).wait()
        compute_scatter(srcv0, dstv0, ewv0, rowsv0)

        @pl.when(jj < _NC1 // 2 - 1)
        def _():
            off2 = base_w + (2 * jj + 2) * _CH1
            load_idx(off2, srcv0, dstv0, ewv0)
            pltpu.async_copy(xe_hbm.at[srcv0], rowsv0, sem0)
        g1.wait()
        compute_scatter(srcv1, dstv1, ewv1, rowsv1)
        return c

    lax.fori_loop(0, _NC1 // 2, body, 0)

    plsc.subcore_barrier()
    pltpu.sync_copy(acc_sh.at[pl.ds(sid * 625, 625)],
                    acc_hbm.at[cid, pl.ds(sid * 625, 625)])

    @pl.when(sid < 15)
    def _():
        pltpu.sync_copy(pres_sh.at[pl.ds(sid * 640, 640)],
                        pres_hbm.at[cid, pl.ds(sid * 640, 640)])

    @pl.when(sid == 15)
    def _():
        pltpu.sync_copy(pres_sh.at[pl.ds(9600, 400)],
                        pres_hbm.at[cid, pl.ds(9600, 400)])


_CH3 = 160
_NFULL3 = (_E // 32) // _CH3
_TAIL3 = (_E // 32) - _NFULL3 * _CH3


@functools.partial(
    pl.kernel, mesh=_MESH, compiler_params=_SCP,
    out_type=jax.ShapeDtypeStruct((2, _N, _C), jnp.float32),
    scratch_types=[
        pltpu.VMEM((_CH3,), jnp.int32),
        pltpu.VMEM((_CH3,), jnp.int32),
        pltpu.VMEM((_CH3,), jnp.float32),
        pltpu.VMEM((_CH3, _C), jnp.float32),
        pltpu.VMEM((_CH3,), jnp.int32),
        pltpu.VMEM((_CH3,), jnp.int32),
        pltpu.VMEM((_CH3,), jnp.float32),
        pltpu.VMEM((_CH3, _C), jnp.float32),
        pltpu.VMEM_SHARED((_N, _C), jnp.float32),
        pltpu.SemaphoreType.DMA,
        pltpu.SemaphoreType.DMA,
        pltpu.SemaphoreType.DMA,
        pltpu.SemaphoreType.DMA,
    ],
)
def _k3(h1_hbm, ei_hbm, ew_hbm, acc_hbm,
        srcv0, dstv0, ewv0, rowsv0, srcv1, dstv1, ewv1, rowsv1,
        acc_sh, sem0, sem1, ssem0, ssem1):
    cid = lax.axis_index("c")
    sid = lax.axis_index("s")
    w = _wid()
    def zr(i, c):
        for jj in range(8):
            rowsv0[i, pl.ds(jj * 16, 16)] = _ZF()
        return c
    lax.fori_loop(0, _CH3, zr, 0)
    for z in range(3):
        pltpu.sync_copy(rowsv0, acc_sh.at[pl.ds(sid * 625 + z * _CH3, _CH3)])
    pltpu.sync_copy(rowsv0.at[pl.ds(0, 625 - 3 * _CH3)],
                    acc_sh.at[pl.ds(sid * 625 + 3 * _CH3, 625 - 3 * _CH3)])
    plsc.subcore_barrier()

    base_w = w * (_E // 32)

    def load_idx(off, srcv, dstv, ewv, n=_CH3):
        pltpu.sync_copy(ei_hbm.at[0, pl.ds(off, n)], srcv.at[pl.ds(0, n)])
        pltpu.sync_copy(ei_hbm.at[1, pl.ds(off, n)], dstv.at[pl.ds(0, n)])
        pltpu.sync_copy(ew_hbm.at[pl.ds(off, n)], ewv.at[pl.ds(0, n)])

    def compute(ewv, rowsv):
        def mul(r, c):
            sc = plsc.load_gather(ewv, [_ZI() + r])
            for jj in range(8):
                rowsv[r, pl.ds(jj * 16, 16)] = (
                    rowsv[r, pl.ds(jj * 16, 16)] * sc)
            return c
        lax.fori_loop(0, _CH3, mul, 0)

    load_idx(base_w, srcv0, dstv0, ewv0)
    pltpu.async_copy(h1_hbm.at[srcv0], rowsv0, sem0)

    def body(jj, c):
        @pl.when(jj > 0)
        def _():
            pltpu.make_async_copy(rowsv1, acc_sh.at[dstv1], ssem1).wait()
        off1 = base_w + (2 * jj + 1) * _CH3
        load_idx(off1, srcv1, dstv1, ewv1)
        g1 = pltpu.async_copy(h1_hbm.at[srcv1], rowsv1, sem1)
        pltpu.make_async_copy(h1_hbm.at[srcv0], rowsv0, sem0).wait()
        compute(ewv0, rowsv0)
        s0 = pltpu.async_copy(rowsv0, acc_sh.at[dstv0], ssem0, add=True)

        g1.wait()
        compute(ewv1, rowsv1)
        s0.wait()

        @pl.when(jj < _NFULL3 // 2 - 1)
        def _():
            off2 = base_w + (2 * jj + 2) * _CH3
            load_idx(off2, srcv0, dstv0, ewv0)
            pltpu.async_copy(h1_hbm.at[srcv0], rowsv0, sem0)
        pltpu.async_copy(rowsv1, acc_sh.at[dstv1], ssem1, add=True)
        return c

    lax.fori_loop(0, _NFULL3 // 2, body, 0)
    pltpu.make_async_copy(rowsv1, acc_sh.at[dstv1], ssem1).wait()

    toff = base_w + _NFULL3 * _CH3
    load_idx(toff, srcv0, dstv0, ewv0, n=_TAIL3)
    pltpu.async_copy(h1_hbm.at[srcv0.at[pl.ds(0, _TAIL3)]],
                     rowsv0.at[pl.ds(0, _TAIL3)], sem0).wait()

    def mult(r, c):
        sc = plsc.load_gather(ewv0, [_ZI() + r])
        for jj in range(8):
            rowsv0[r, pl.ds(jj * 16, 16)] = rowsv0[r, pl.ds(jj * 16, 16)] * sc
        return c
    lax.fori_loop(0, _TAIL3, mult, 0)
    pltpu.sync_copy(rowsv0.at[pl.ds(0, _TAIL3)],
                    acc_sh.at[dstv0.at[pl.ds(0, _TAIL3)]], add=True)

    plsc.subcore_barrier()
    pltpu.sync_copy(acc_sh.at[pl.ds(sid * 625, 625)],
                    acc_hbm.at[cid, pl.ds(sid * 625, 625)])


def _k2_body(acc_ref, x_ref, w1_ref, w2_ref, w3_ref, h1_ref):
    acc = acc_ref[0] + acc_ref[1]
    deg = acc[:, 4:5]
    x = x_ref[...]
    t1 = jnp.dot(acc, w1_ref[...], preferred_element_type=jnp.float32)
    t2 = jnp.dot(x, w2_ref[...], preferred_element_type=jnp.float32)
    t3 = jnp.dot(x, w3_ref[...], preferred_element_type=jnp.float32)
    h1_ref[...] = jnp.maximum(t1 - deg * t2 + t3, 0.0)


_k2 = pl.pallas_call(
    _k2_body,
    grid=(5,),
    in_specs=[
        pl.BlockSpec((2, 2000, 16), lambda i: (0, i, 0)),
        pl.BlockSpec((2000, 16), lambda i: (i, 0)),
        pl.BlockSpec((16, _C), lambda i: (0, 0)),
        pl.BlockSpec((16, _C), lambda i: (0, 0)),
        pl.BlockSpec((16, _C), lambda i: (0, 0)),
    ],
    out_specs=pl.BlockSpec((2000, _C), lambda i: (i, 0)),
    out_shape=jax.ShapeDtypeStruct((_N, _C), jnp.float32),
)


def _k4_body(acc2_ref, acc1_ref, h1_ref, w1_ref, w2_ref, w3_ref, b1_ref,
             b3_ref, h_ref):
    z2 = acc2_ref[0] + acc2_ref[1]
    deg = acc1_ref[0][:, 4:5] + acc1_ref[1][:, 4:5]
    h1 = h1_ref[...]
    t1 = jnp.dot(z2, w1_ref[...], preferred_element_type=jnp.float32)
    t2 = jnp.dot(h1, w2_ref[...], preferred_element_type=jnp.float32)
    t3 = jnp.dot(h1, w3_ref[...], preferred_element_type=jnp.float32)
    h_ref[...] = t1 + deg * b1_ref[...] - deg * t2 + t3 + b3_ref[...]


_k4 = pl.pallas_call(
    _k4_body,
    grid=(5,),
    in_specs=[
        pl.BlockSpec((2, 2000, _C), lambda i: (0, i, 0)),
        pl.BlockSpec((2, 2000, 16), lambda i: (0, i, 0)),
        pl.BlockSpec((2000, _C), lambda i: (i, 0)),
        pl.BlockSpec((_C, _C), lambda i: (0, 0)),
        pl.BlockSpec((_C, _C), lambda i: (0, 0)),
        pl.BlockSpec((_C, _C), lambda i: (0, 0)),
        pl.BlockSpec((1, _C), lambda i: (0, 0)),
        pl.BlockSpec((1, _C), lambda i: (0, 0)),
    ],
    out_specs=pl.BlockSpec((2000, _C), lambda i: (i, 0)),
    out_shape=jax.ShapeDtypeStruct((_N, _C), jnp.float32),
)


_W5 = _E // 16
_S5 = _W5 // 16
_RDX = 256
_KCH = 2000


@functools.partial(
    pl.kernel, mesh=_MESH, compiler_params=_SCP,
    out_type=(jax.ShapeDtypeStruct((_R,), jnp.float32),
              jax.ShapeDtypeStruct((_R,), jnp.float32),
              jax.ShapeDtypeStruct((_R,), jnp.int32),
              jax.ShapeDtypeStruct((_R,), jnp.int32),
              jax.ShapeDtypeStruct((_N,), jnp.float32)),
    scratch_types=[
        pltpu.VMEM((_W5,), jnp.float32),
        pltpu.VMEM((_W5,), jnp.int32),
        pltpu.VMEM((_W5,), jnp.int32),
        pltpu.VMEM((16 * _RDX,), jnp.int32),
        pltpu.VMEM((16 * _RDX,), jnp.int32),
        pltpu.VMEM((_RDX,), jnp.int32),
        pltpu.VMEM((_KCH,), jnp.float32),
        pltpu.VMEM((_KCH,), jnp.int32),
        pltpu.VMEM((_KCH,), jnp.int32),
        pltpu.VMEM((_KCH,), jnp.float32),
        pltpu.VMEM((640,), jnp.float32),
        pltpu.VMEM_SHARED((_E,), jnp.int32),
        pltpu.VMEM_SHARED((_E,), jnp.int32),
        pltpu.VMEM_SHARED((16 * _RDX,), jnp.int32),
        pltpu.VMEM_SHARED((_N,), jnp.float32),
        pltpu.SemaphoreType.DMA,
    ],
)
def _k5(score_hbm, ew_hbm, src_hbm, dst_hbm,
        rew_hbm, rea_hbm, rsrc_hbm, rdst_hbm, presr_hbm,
        keysf, valsv, posv, histv, Tallv, Tv, s2k, i2k, j2k, ones2k,
        zi640, A_v, B_v, Tall_sh, pres_r_sh, sem):
    cid = lax.axis_index("c")
    sid = lax.axis_index("s")
    lanes = _LANES()

    @pl.when(cid == 0)
    def _():
        base = sid * _W5
        _zero_vmem(zi640, 640, jnp.float32)

        @pl.when(sid < 15)
        def _():
            pltpu.sync_copy(zi640, pres_r_sh.at[pl.ds(sid * 640, 640)])

        @pl.when(sid == 15)
        def _():
            pltpu.sync_copy(zi640.at[pl.ds(0, 400)],
                            pres_r_sh.at[pl.ds(9600, 400)])

        def o1(i, c):
            ones2k[pl.ds(i * 16, 16)] = _ZF() + 1.0
            return c
        lax.fori_loop(0, _KCH // 16, o1, 0)

        for p in range(4):
            shift = 8 * p
            if p % 2 == 0:
                src_v, dst_v = A_v, B_v
            else:
                src_v, dst_v = B_v, A_v
            if p == 0:
                def iv(i, c):
                    valsv[pl.ds(i * 16, 16)] = base + lanes * _S5 + i
                    return c
                lax.fori_loop(0, _S5, iv, 0)
            else:
                pltpu.sync_copy(src_v.at[pl.ds(base, _W5)], posv)

                def tv(i, c):
                    valsv[pl.ds(i * 16, 16)] = plsc.load_gather(
                        posv, [lanes * _S5 + i])
                    return c
                lax.fori_loop(0, _S5, tv, 0)
            pltpu.async_copy(score_hbm.at[valsv], keysf, sem).wait()

            def mk(i, c):
                sl = pl.ds(i * 16, 16)
                keysf[sl] = plsc.bitcast(
                    _KEYMAX - plsc.bitcast(keysf[sl], jnp.int32), jnp.float32)
                return c
            lax.fori_loop(0, _S5, mk, 0)

            def zh(i, c):
                histv[pl.ds(i * 16, 16)] = _ZI()
                return c
            lax.fori_loop(0, _RDX, zh, 0)

            def ha(i, c):
                sl = pl.ds(i * 16, 16)
                kv = plsc.bitcast(keysf[sl], jnp.int32)
                d = jnp.bitwise_and(lax.shift_right_logical(kv, shift),
                                    _RDX - 1)
                fl = lanes * _RDX + d
                cnt = plsc.load_gather(histv, [fl])
                plsc.store_scatter(histv, [fl], cnt + 1)
                posv[sl] = cnt
                return c
            lax.fori_loop(0, _S5, ha, 0)

            def wt(j, c):
                acc = _ZI()
                for l in range(16):
                    acc = acc + histv[pl.ds(l * _RDX + j * 16, 16)]
                Tv[pl.ds(j * 16, 16)] = acc
                return c
            lax.fori_loop(0, _RDX // 16, wt, 0)
            pltpu.sync_copy(Tv, Tall_sh.at[pl.ds(sid * _RDX, _RDX)])
            plsc.subcore_barrier()

            pltpu.sync_copy(Tall_sh, Tallv)

            def pb(j, carry):
                G = _ZI()
                Wp = _ZI()
                for wk in range(16):
                    s_w = Tallv[pl.ds(wk * _RDX + j * 16, 16)]
                    G = G + s_w
                    Wp = Wp + s_w * jnp.where(wk < sid, 1, 0)
                cs = plsc.cumsum(G)
                Px = carry + cs - G
                run = Px + Wp
                for l in range(16):
                    sl_ = pl.ds(l * _RDX + j * 16, 16)
                    tmp = histv[sl_]
                    histv[sl_] = run
                    run = run + tmp
                return carry + jnp.sum(G)
            lax.fori_loop(0, _RDX // 16, pb, jnp.int32(0))

            def pc(i, c):
                sl = pl.ds(i * 16, 16)
                kv = plsc.bitcast(keysf[sl], jnp.int32)
                d = jnp.bitwise_and(lax.shift_right_logical(kv, shift),
                                    _RDX - 1)
                base_off = plsc.load_gather(histv, [lanes * _RDX + d])
                posv[sl] = posv[sl] + base_off
                return c
            lax.fori_loop(0, _S5, pc, 0)

            pltpu.sync_copy(valsv, dst_v.at[posv])
            plsc.subcore_barrier()

        base2 = sid * (_R // 16)
        for q in range((_R // 16) // _KCH):
            off2 = base2 + q * _KCH
            pltpu.sync_copy(A_v.at[pl.ds(off2, _KCH)], i2k)
            pltpu.async_copy(score_hbm.at[i2k], s2k, sem).wait()
            pltpu.sync_copy(s2k, rew_hbm.at[pl.ds(off2, _KCH)])
            pltpu.async_copy(ew_hbm.at[i2k], s2k, sem).wait()
            pltpu.sync_copy(s2k, rea_hbm.at[pl.ds(off2, _KCH)])
            pltpu.async_copy(src_hbm.at[i2k], j2k, sem).wait()
            pltpu.sync_copy(j2k, rsrc_hbm.at[pl.ds(off2, _KCH)])
            pltpu.sync_copy(ones2k, pres_r_sh.at[j2k], add=True)
            pltpu.async_copy(dst_hbm.at[i2k], j2k, sem).wait()
            pltpu.sync_copy(j2k, rdst_hbm.at[pl.ds(off2, _KCH)])
            pltpu.sync_copy(ones2k, pres_r_sh.at[j2k], add=True)

        plsc.subcore_barrier()

        @pl.when(sid < 15)
        def _():
            pltpu.sync_copy(pres_r_sh.at[pl.ds(sid * 640, 640)],
                            presr_hbm.at[pl.ds(sid * 640, 640)])

        @pl.when(sid == 15)
        def _():
            pltpu.sync_copy(pres_r_sh.at[pl.ds(9600, 400)],
                            presr_hbm.at[pl.ds(9600, 400)])


def _k7_body(pf_ref, pr_ref, nif_ref, nir_ref):
    r128 = lax.broadcasted_iota(jnp.int32, (_C, _C), 0)
    c128 = lax.broadcasted_iota(jnp.int32, (_C, _C), 1)
    Lm = (r128 <= c128).astype(jnp.float32)
    Jm = jnp.ones((_C, _C), jnp.float32)
    r80 = lax.broadcasted_iota(jnp.int32, (80, 80), 0)
    c80 = lax.broadcasted_iota(jnp.int32, (80, 80), 1)
    SL = (c80 < r80).astype(jnp.float32)

    def ranks(p):
        pfl = (p > 0).astype(jnp.float32)
        incl = (jnp.dot(pfl, Lm, preferred_element_type=jnp.float32)
                + jnp.dot(SL, jnp.dot(pfl, Jm,
                                      preferred_element_type=jnp.float32),
                          preferred_element_type=jnp.float32))
        return jnp.where(pfl > 0, incl - pfl, -1.0).astype(jnp.int32)

    nif_ref[...] = ranks(pf_ref[0] + pf_ref[1])
    nir_ref[...] = ranks(pr_ref[...])


_k7 = pl.pallas_call(
    _k7_body,
    in_specs=[pl.BlockSpec((2, 80, _C), lambda: (0, 0, 0)),
              pl.BlockSpec((80, _C), lambda: (0, 0))],
    out_specs=(pl.BlockSpec((80, _C), lambda: (0, 0)),
               pl.BlockSpec((80, _C), lambda: (0, 0))),
    out_shape=(jax.ShapeDtypeStruct((80, _C), jnp.int32),
               jax.ShapeDtypeStruct((80, _C), jnp.int32)),
)


_NCH = 400
_ECH = _E // 32
_RCH = _R // 32


@functools.partial(
    pl.kernel, mesh=_MESH, compiler_params=_SCP,
    out_type=(jax.ShapeDtypeStruct((_N, _C), jnp.float32),
              jax.ShapeDtypeStruct((_N, _C), jnp.float32),
              jax.ShapeDtypeStruct((_N,), jnp.int32),
              jax.ShapeDtypeStruct((_N,), jnp.int32),
              jax.ShapeDtypeStruct((2, _R), jnp.int32),
              jax.ShapeDtypeStruct((2, _E), jnp.int32)),
    scratch_types=[
        pltpu.VMEM((_NCH,), jnp.int32),
        pltpu.VMEM((_NCH,), jnp.int32),
        pltpu.VMEM((_NCH,), jnp.int32),
        pltpu.VMEM((_NCH, _C), jnp.float32),
        pltpu.VMEM((_ECH,), jnp.int32),
        pltpu.VMEM((_ECH,), jnp.int32),
        pltpu.VMEM((_RCH,), jnp.int32),
        pltpu.VMEM((_RCH,), jnp.int32),
        pltpu.VMEM((_N + 16,), jnp.int32),
        pltpu.VMEM_SHARED((_N + 16,), jnp.int32),
        pltpu.VMEM_SHARED((_N + 16,), jnp.int32),
        pltpu.VMEM_SHARED((_N,), jnp.int32),
        pltpu.VMEM_SHARED((_N,), jnp.int32),
        pltpu.VMEM_SHARED((_N,), jnp.int32),
        pltpu.SemaphoreType.DMA,
    ],
)
def _k8(h_hbm, batch_hbm, nif_hbm, nir_hbm, rsrc_hbm, rdst_hbm, src_hbm,
        dst_hbm, rx_hbm, fx_hbm, rb_hbm, fb_hbm, rei_hbm, fei_hbm,
        nv, posv, valv, rowsv, eb, ob, eb7, ob7, big, sub_f, sub_r, nif_sh,
        nir_sh, batch_sh, sem):
    cid = lax.axis_index("c")
    sid = lax.axis_index("s")
    w = _wid()
    lanes = _LANES()
    @pl.when(sid == 0)
    def _():
        pltpu.sync_copy(nif_hbm, big.at[pl.ds(0, _N)])
        pltpu.sync_copy(big.at[pl.ds(0, _N)], nif_sh)

    @pl.when(sid == 1)
    def _():
        pltpu.sync_copy(nir_hbm, big.at[pl.ds(0, _N)])
        pltpu.sync_copy(big.at[pl.ds(0, _N)], nir_sh)

    @pl.when(sid == 2)
    def _():
        pltpu.sync_copy(batch_hbm, big.at[pl.ds(0, _N)])
        pltpu.sync_copy(big.at[pl.ds(0, _N)], batch_sh)

    @pl.when(sid == 3)
    def _():
        _zero_vmem(big, _N + 16, jnp.int32)
        pltpu.sync_copy(big, sub_f)

    @pl.when(sid == 4)
    def _():
        _zero_vmem(big, _N + 16, jnp.int32)
        pltpu.sync_copy(big, sub_r)

    plsc.subcore_barrier()

    for br in range(2):
        ni_sh = nif_sh if br == 0 else nir_sh
        sub_sh = sub_f if br == 0 else sub_r
        for c in range(_N // _NCH):
            @pl.when(c % 16 == sid)
            def _(c=c, ni_sh=ni_sh, sub_sh=sub_sh):
                pltpu.sync_copy(ni_sh.at[pl.ds(c * _NCH, _NCH)], nv)

                def bld(i, cc):
                    pos = nv[pl.ds(i * 16, 16)]
                    posv[pl.ds(i * 16, 16)] = jnp.where(
                        pos < 0, _N + lanes, pos)
                    valv[pl.ds(i * 16, 16)] = c * _NCH + i * 16 + lanes
                    return cc
                lax.fori_loop(0, _NCH // 16, bld, 0)
                pltpu.sync_copy(valv, sub_sh.at[posv])
    plsc.subcore_barrier()

    for t in range(2):
        if t == 0:
            cidx = w
        else:
            cidx = 32 + sid + 9 * cid

        @pl.when(jnp.logical_and(t == 0 or sid < 9, cidx < _N // _NCH))
        def _(cidx=cidx):
            off = cidx * _NCH
            pltpu.sync_copy(sub_r.at[pl.ds(off, _NCH)], nv)
            pltpu.async_copy(h_hbm.at[nv], rowsv, sem).wait()
            pltpu.sync_copy(rowsv, rx_hbm.at[pl.ds(off, _NCH)])
            pltpu.sync_copy(batch_sh.at[nv], posv)
            pltpu.sync_copy(posv, rb_hbm.at[pl.ds(off, _NCH)])

        @pl.when(jnp.logical_and(t == 0 or sid < 9,
                                 jnp.logical_and(cidx >= _N // _NCH,
                                                 cidx < 2 * (_N // _NCH))))
        def _(cidx=cidx):
            off = (cidx - _N // _NCH) * _NCH
            pltpu.sync_copy(sub_f.at[pl.ds(off, _NCH)], nv)
            pltpu.async_copy(h_hbm.at[nv], rowsv, sem).wait()
            pltpu.sync_copy(rowsv, fx_hbm.at[pl.ds(off, _NCH)])
            pltpu.sync_copy(batch_sh.at[nv], posv)
            pltpu.sync_copy(posv, fb_hbm.at[pl.ds(off, _NCH)])

    eoff = w * _ECH
    pltpu.sync_copy(src_hbm.at[pl.ds(eoff, _ECH)], eb)
    pltpu.sync_copy(nif_sh.at[eb], ob)
    pltpu.sync_copy(ob, fei_hbm.at[0, pl.ds(eoff, _ECH)])
    pltpu.sync_copy(dst_hbm.at[pl.ds(eoff, _ECH)], eb)
    pltpu.sync_copy(nif_sh.at[eb], ob)
    pltpu.sync_copy(ob, fei_hbm.at[1, pl.ds(eoff, _ECH)])
    roff = w * _RCH
    pltpu.sync_copy(rsrc_hbm.at[pl.ds(roff, _RCH)], eb7)
    pltpu.sync_copy(nir_sh.at[eb7], ob7)
    pltpu.sync_copy(ob7, rei_hbm.at[0, pl.ds(roff, _RCH)])
    pltpu.sync_copy(rdst_hbm.at[pl.ds(roff, _RCH)], eb7)
    pltpu.sync_copy(nir_sh.at[eb7], ob7)
    pltpu.sync_copy(ob7, rei_hbm.at[1, pl.ds(roff, _RCH)])


def kernel(x, edge_index, edge_attr, batch, edge_score, c1_w1, c1_b1, c1_w2,
           c1_w3, c1_b3, c2_w1, c2_b1, c2_w2, c2_w3, c2_b3):
    src = edge_index[0]
    dst = edge_index[1]
    ew = edge_attr.reshape(-1)
    xe = jnp.concatenate(
        [x, jnp.ones((_N, 1), jnp.float32), jnp.zeros((_N, 11), jnp.float32)],
        axis=1)

    acc1, pres_f = _k1(xe, edge_index, ew)

    zw = jnp.zeros((16, _C), jnp.float32)
    w1a = zw.at[0:4].set(c1_w1).at[4].set(c1_b1)
    w2a = zw.at[0:4].set(c1_w2)
    w3a = zw.at[0:4].set(c1_w3).at[4].set(c1_b3)
    h1 = _k2(acc1, xe, w1a, w2a, w3a)

    acc2 = _k3(h1, edge_index, ew)
    h = _k4(acc2, acc1, h1, c2_w1, c2_w2, c2_w3, c2_b1[None, :],
            c2_b3[None, :])

    r_ew, r_ea, r_src, r_dst, pres_r = _k5(edge_score, ew, src, dst)

    pf = jnp.pad(pres_f, ((0, 0), (0, 10240 - _N))).reshape(2, 80, _C)
    pr = jnp.pad(pres_r, (0, 10240 - _N)).reshape(80, _C)
    nif80, nir80 = _k7(pf, pr)
    nif = nif80.reshape(-1)[:_N]
    nir = nir80.reshape(-1)[:_N]

    r_x, f_x, r_batch, f_batch, r_ei2, f_ei2 = _k8(
        h, batch, nif, nir, r_src, r_dst, src, dst)

    return ((r_x, r_ei2, r_ea, r_ew, r_batch),
            (f_x, f_ei2, ew, edge_score, f_batch),
            edge_score)


# speedup vs baseline: 19.7970x; 1.0594x over previous
import functools
import jax
import jax.numpy as jnp
from jax import lax
from jax.experimental import pallas as pl
from jax.experimental.pallas import tpu as pltpu, tpu_sc as plsc

_N = 10000
_E = 320000
_C = 128
_R = 224000
_KEYMAX = 0x3F7FFFFF

_MESH = plsc.VectorSubcoreMesh(core_axis_name="c", subcore_axis_name="s")
_SCP = pltpu.CompilerParams(needs_layout_passes=False, use_tc_tiling_on_sc=False)

_LANES = lambda: lax.iota(jnp.int32, 16)
_ZI = lambda: jnp.zeros((16,), jnp.int32)
_ZF = lambda: jnp.zeros((16,), jnp.float32)


def _wid():
    return lax.axis_index("c") * 16 + lax.axis_index("s")


def _zero_vmem(ref, n, dtype):
    z = jnp.zeros((16,), dtype)

    def body(i, c):
        ref[pl.ds(i * 16, 16)] = z
        return c

    lax.fori_loop(0, n // 16, body, 0)


_CH1 = 1000


@functools.partial(
    pl.kernel, mesh=_MESH, compiler_params=_SCP,
    out_type=(jax.ShapeDtypeStruct((2, _N, 16), jnp.float32),
              jax.ShapeDtypeStruct((2, _N), jnp.float32)),
    scratch_types=[
        pltpu.VMEM((_CH1,), jnp.int32),
        pltpu.VMEM((_CH1,), jnp.int32),
        pltpu.VMEM((_CH1,), jnp.float32),
        pltpu.VMEM((_CH1, 16), jnp.float32),
        pltpu.VMEM((_CH1,), jnp.int32),
        pltpu.VMEM((_CH1,), jnp.int32),
        pltpu.VMEM((_CH1,), jnp.float32),
        pltpu.VMEM((_CH1, 16), jnp.float32),
        pltpu.VMEM((_CH1,), jnp.float32),
        pltpu.VMEM((640, 16), jnp.float32),
        pltpu.VMEM((640,), jnp.float32),
        pltpu.VMEM_SHARED((_N, 16), jnp.float32),
        pltpu.VMEM_SHARED((_N,), jnp.float32),
        pltpu.SemaphoreType.DMA,
        pltpu.SemaphoreType.DMA,
    ],
)
def _k1(xe_hbm, ei_hbm, ew_hbm, acc_hbm, pres_hbm,
        srcv0, dstv0, ewv0, rowsv0, srcv1, dstv1, ewv1, rowsv1,
        onesv, zrows, zi640, acc_sh, pres_sh, sem0, sem1):
    cid = lax.axis_index("c")
    sid = lax.axis_index("s")
    w = _wid()
    lanes = _LANES()
    def zr(i, c):
        zrows[i, :] = _ZF()
        return c
    lax.fori_loop(0, 640, zr, 0)
    _zero_vmem(zi640, 640, jnp.float32)
    pltpu.sync_copy(zrows.at[pl.ds(0, 625)], acc_sh.at[pl.ds(sid * 625, 625)])

    @pl.when(sid < 15)
    def _():
        pltpu.sync_copy(zi640, pres_sh.at[pl.ds(sid * 640, 640)])

    @pl.when(sid == 15)
    def _():
        pltpu.sync_copy(zi640.at[pl.ds(0, 400)], pres_sh.at[pl.ds(9600, 400)])

    def o1(i, c):
        onesv[pl.ds(i * 16, 16)] = _ZF() + 1.0
        return c
    lax.fori_loop(0, _CH1 // 16, o1, 0)
    plsc.subcore_barrier()

    base_w = w * (_E // 32)
    _NC1 = (_E // 32) // _CH1

    def load_idx(off, srcv, dstv, ewv):
        pltpu.sync_copy(ei_hbm.at[0, pl.ds(off, _CH1)], srcv)
        pltpu.sync_copy(ei_hbm.at[1, pl.ds(off, _CH1)], dstv)
        pltpu.sync_copy(ew_hbm.at[pl.ds(off, _CH1)], ewv)

    def compute_scatter(srcv, dstv, ewv, rowsv):
        def mul(r, c):
            sc = plsc.load_gather(ewv, [_ZI() + r])
            rowsv[r, :] = rowsv[r, :] * sc
            return c
        lax.fori_loop(0, _CH1, mul, 0)
        pltpu.sync_copy(rowsv, acc_sh.at[dstv], add=True)
        pltpu.sync_copy(onesv, pres_sh.at[srcv], add=True)
        pltpu.sync_copy(onesv, pres_sh.at[dstv], add=True)

    load_idx(base_w, srcv0, dstv0, ewv0)
    pltpu.async_copy(xe_hbm.at[srcv0], rowsv0, sem0)

    def body(jj, c):
        off1 = base_w + (2 * jj + 1) * _CH1
        load_idx(off1, srcv1, dstv1, ewv1)
        g1 = pltpu.async_copy(xe_hbm.at[srcv1], rowsv1, sem1)
        pltpu.make_async_copy(xe_hbm.at[srcv0], rowsv0, sem0).wait()
        compute_scatter(srcv0, dstv0, ewv0, rowsv0)

        @pl.when(jj < _NC1 // 2 - 1)
        def _():
            off2 = base_w + (2 * jj + 2) * _CH1
            load_idx(off2, srcv0, dstv0, ewv0)
            pltpu.async_copy(xe_hbm.at[srcv0], rowsv0, sem0)
        g1.wait()
        compute_scatter(srcv1, dstv1, ewv1, rowsv1)
        return c

    lax.fori_loop(0, _NC1 // 2, body, 0)

    plsc.subcore_barrier()
    pltpu.sync_copy(acc_sh.at[pl.ds(sid * 625, 625)],
                    acc_hbm.at[cid, pl.ds(sid * 625, 625)])

    @pl.when(sid < 15)
    def _():
        pltpu.sync_copy(pres_sh.at[pl.ds(sid * 640, 640)],
                        pres_hbm.at[cid, pl.ds(sid * 640, 640)])

    @pl.when(sid == 15)
    def _():
        pltpu.sync_copy(pres_sh.at[pl.ds(9600, 400)],
                        pres_hbm.at[cid, pl.ds(9600, 400)])


_CH3 = 160
_NFULL3 = (_E // 32) // _CH3
_TAIL3 = (_E // 32) - _NFULL3 * _CH3


@functools.partial(
    pl.kernel, mesh=_MESH, compiler_params=_SCP,
    out_type=jax.ShapeDtypeStruct((2, _N, _C), jnp.float32),
    scratch_types=[
        pltpu.VMEM((_CH3,), jnp.int32),
        pltpu.VMEM((_CH3,), jnp.int32),
        pltpu.VMEM((_CH3,), jnp.float32),
        pltpu.VMEM((_CH3, _C), jnp.float32),
        pltpu.VMEM((_CH3,), jnp.int32),
        pltpu.VMEM((_CH3,), jnp.int32),
        pltpu.VMEM((_CH3,), jnp.float32),
        pltpu.VMEM((_CH3, _C), jnp.float32),
        pltpu.VMEM_SHARED((_N, _C), jnp.float32),
        pltpu.SemaphoreType.DMA,
        pltpu.SemaphoreType.DMA,
        pltpu.SemaphoreType.DMA,
        pltpu.SemaphoreType.DMA,
    ],
)
def _k3(h1_hbm, ei_hbm, ew_hbm, acc_hbm,
        srcv0, dstv0, ewv0, rowsv0, srcv1, dstv1, ewv1, rowsv1,
        acc_sh, sem0, sem1, ssem0, ssem1):
    cid = lax.axis_index("c")
    sid = lax.axis_index("s")
    w = _wid()
    def zr(i, c):
        for jj in range(8):
            rowsv0[i, pl.ds(jj * 16, 16)] = _ZF()
        return c
    lax.fori_loop(0, _CH3, zr, 0)
    for z in range(3):
        pltpu.sync_copy(rowsv0, acc_sh.at[pl.ds(sid * 625 + z * _CH3, _CH3)])
    pltpu.sync_copy(rowsv0.at[pl.ds(0, 625 - 3 * _CH3)],
                    acc_sh.at[pl.ds(sid * 625 + 3 * _CH3, 625 - 3 * _CH3)])
    plsc.subcore_barrier()

    base_w = w * (_E // 32)

    def load_idx(off, srcv, dstv, ewv, n=_CH3):
        pltpu.sync_copy(ei_hbm.at[0, pl.ds(off, n)], srcv.at[pl.ds(0, n)])
        pltpu.sync_copy(ei_hbm.at[1, pl.ds(off, n)], dstv.at[pl.ds(0, n)])
        pltpu.sync_copy(ew_hbm.at[pl.ds(off, n)], ewv.at[pl.ds(0, n)])

    def compute(ewv, rowsv):
        def mul(r, c):
            sc = plsc.load_gather(ewv, [_ZI() + r])
            for jj in range(8):
                rowsv[r, pl.ds(jj * 16, 16)] = (
                    rowsv[r, pl.ds(jj * 16, 16)] * sc)
            return c
        lax.fori_loop(0, _CH3, mul, 0)

    load_idx(base_w, srcv0, dstv0, ewv0)
    pltpu.async_copy(h1_hbm.at[srcv0], rowsv0, sem0)

    def body(jj, c):
        @pl.when(jj > 0)
        def _():
            pltpu.make_async_copy(rowsv1, acc_sh.at[dstv1], ssem1).wait()
        off1 = base_w + (2 * jj + 1) * _CH3
        load_idx(off1, srcv1, dstv1, ewv1)
        g1 = pltpu.async_copy(h1_hbm.at[srcv1], rowsv1, sem1)
        pltpu.make_async_copy(h1_hbm.at[srcv0], rowsv0, sem0).wait()
        compute(ewv0, rowsv0)
        s0 = pltpu.async_copy(rowsv0, acc_sh.at[dstv0], ssem0, add=True)

        g1.wait()
        compute(ewv1, rowsv1)
        s0.wait()

        @pl.when(jj < _NFULL3 // 2 - 1)
        def _():
            off2 = base_w + (2 * jj + 2) * _CH3
            load_idx(off2, srcv0, dstv0, ewv0)
            pltpu.async_copy(h1_hbm.at[srcv0], rowsv0, sem0)
        pltpu.async_copy(rowsv1, acc_sh.at[dstv1], ssem1, add=True)
        return c

    lax.fori_loop(0, _NFULL3 // 2, body, 0)
    pltpu.make_async_copy(rowsv1, acc_sh.at[dstv1], ssem1).wait()

    toff = base_w + _NFULL3 * _CH3
    load_idx(toff, srcv0, dstv0, ewv0, n=_TAIL3)
    pltpu.async_copy(h1_hbm.at[srcv0.at[pl.ds(0, _TAIL3)]],
                     rowsv0.at[pl.ds(0, _TAIL3)], sem0).wait()

    def mult(r, c):
        sc = plsc.load_gather(ewv0, [_ZI() + r])
        for jj in range(8):
            rowsv0[r, pl.ds(jj * 16, 16)] = rowsv0[r, pl.ds(jj * 16, 16)] * sc
        return c
    lax.fori_loop(0, _TAIL3, mult, 0)
    pltpu.sync_copy(rowsv0.at[pl.ds(0, _TAIL3)],
                    acc_sh.at[dstv0.at[pl.ds(0, _TAIL3)]], add=True)

    plsc.subcore_barrier()
    pltpu.sync_copy(acc_sh.at[pl.ds(sid * 625, 625)],
                    acc_hbm.at[cid, pl.ds(sid * 625, 625)])


def _k2_body(acc_ref, x_ref, w1_ref, w2_ref, w3_ref, h1_ref):
    acc = acc_ref[0] + acc_ref[1]
    deg = acc[:, 4:5]
    x = x_ref[...]
    t1 = jnp.dot(acc, w1_ref[...], preferred_element_type=jnp.float32)
    t2 = jnp.dot(x, w2_ref[...], preferred_element_type=jnp.float32)
    t3 = jnp.dot(x, w3_ref[...], preferred_element_type=jnp.float32)
    h1_ref[...] = jnp.maximum(t1 - deg * t2 + t3, 0.0)


_k2 = pl.pallas_call(
    _k2_body,
    grid=(5,),
    in_specs=[
        pl.BlockSpec((2, 2000, 16), lambda i: (0, i, 0)),
        pl.BlockSpec((2000, 16), lambda i: (i, 0)),
        pl.BlockSpec((16, _C), lambda i: (0, 0)),
        pl.BlockSpec((16, _C), lambda i: (0, 0)),
        pl.BlockSpec((16, _C), lambda i: (0, 0)),
    ],
    out_specs=pl.BlockSpec((2000, _C), lambda i: (i, 0)),
    out_shape=jax.ShapeDtypeStruct((_N, _C), jnp.float32),
)


def _k4_body(acc2_ref, acc1_ref, h1_ref, w1_ref, w2_ref, w3_ref, b1_ref,
             b3_ref, h_ref):
    z2 = acc2_ref[0] + acc2_ref[1]
    deg = acc1_ref[0][:, 4:5] + acc1_ref[1][:, 4:5]
    h1 = h1_ref[...]
    t1 = jnp.dot(z2, w1_ref[...], preferred_element_type=jnp.float32)
    t2 = jnp.dot(h1, w2_ref[...], preferred_element_type=jnp.float32)
    t3 = jnp.dot(h1, w3_ref[...], preferred_element_type=jnp.float32)
    h_ref[...] = t1 + deg * b1_ref[...] - deg * t2 + t3 + b3_ref[...]


_k4 = pl.pallas_call(
    _k4_body,
    grid=(5,),
    in_specs=[
        pl.BlockSpec((2, 2000, _C), lambda i: (0, i, 0)),
        pl.BlockSpec((2, 2000, 16), lambda i: (0, i, 0)),
        pl.BlockSpec((2000, _C), lambda i: (i, 0)),
        pl.BlockSpec((_C, _C), lambda i: (0, 0)),
        pl.BlockSpec((_C, _C), lambda i: (0, 0)),
        pl.BlockSpec((_C, _C), lambda i: (0, 0)),
        pl.BlockSpec((1, _C), lambda i: (0, 0)),
        pl.BlockSpec((1, _C), lambda i: (0, 0)),
    ],
    out_specs=pl.BlockSpec((2000, _C), lambda i: (i, 0)),
    out_shape=jax.ShapeDtypeStruct((_N, _C), jnp.float32),
)


_W5 = _E // 16
_S5 = _W5 // 16
_RDX = 1024
_KCH = 1400


@functools.partial(
    pl.kernel, mesh=_MESH, compiler_params=_SCP,
    out_type=(jax.ShapeDtypeStruct((_R,), jnp.float32),
              jax.ShapeDtypeStruct((_R,), jnp.float32),
              jax.ShapeDtypeStruct((_R,), jnp.int32),
              jax.ShapeDtypeStruct((_R,), jnp.int32),
              jax.ShapeDtypeStruct((_N,), jnp.float32)),
    scratch_types=[
        pltpu.VMEM((_W5,), jnp.float32),
        pltpu.VMEM((_W5,), jnp.int32),
        pltpu.VMEM((_W5,), jnp.int32),
        pltpu.VMEM((16 * _RDX,), jnp.int32),
        pltpu.VMEM((4096,), jnp.int32),
        pltpu.VMEM((_RDX,), jnp.int32),
        pltpu.VMEM((_RDX,), jnp.int32),
        pltpu.VMEM((_KCH,), jnp.float32),
        pltpu.VMEM((_KCH,), jnp.int32),
        pltpu.VMEM((_KCH,), jnp.int32),
        pltpu.VMEM((_KCH,), jnp.float32),
        pltpu.VMEM((640,), jnp.float32),
        pltpu.VMEM_SHARED((_E,), jnp.int32),
        pltpu.VMEM_SHARED((_E,), jnp.int32),
        pltpu.VMEM_SHARED((16 * _RDX,), jnp.int32),
        pltpu.VMEM_SHARED((_N,), jnp.float32),
        pltpu.SemaphoreType.DMA,
    ],
)
def _k5(score_hbm, ew_hbm, src_hbm, dst_hbm,
        rew_hbm, rea_hbm, rsrc_hbm, rdst_hbm, presr_hbm,
        keysf, valsv, posv, histv, tbuf, idx1k, Tv, s2k, i2k, j2k, ones2k,
        zi640, A_v, B_v, Tall_sh, pres_r_sh, sem):
    cid = lax.axis_index("c")
    sid = lax.axis_index("s")
    lanes = _LANES()

    @pl.when(cid == 0)
    def _():
        base = sid * _W5
        _zero_vmem(zi640, 640, jnp.float32)

        @pl.when(sid < 15)
        def _():
            pltpu.sync_copy(zi640, pres_r_sh.at[pl.ds(sid * 640, 640)])

        @pl.when(sid == 15)
        def _():
            pltpu.sync_copy(zi640.at[pl.ds(0, 400)],
                            pres_r_sh.at[pl.ds(9600, 400)])

        def o1(i, c):
            ones2k[pl.ds(i * 16, 16)] = _ZF() + 1.0
            return c
        lax.fori_loop(0, _KCH // 16, o1, 0)

        def ix(i, c):
            idx1k[pl.ds(i * 16, 16)] = (i * 16 + lanes) * 16 + sid
            return c
        lax.fori_loop(0, _RDX // 16, ix, 0)

        for p in range(3):
            shift = 10 * p
            if p == 1:
                src_v, dst_v = A_v, B_v
            else:
                src_v, dst_v = B_v, A_v
            if p == 0:
                def iv(i, c):
                    valsv[pl.ds(i * 16, 16)] = base + lanes * _S5 + i
                    return c
                lax.fori_loop(0, _S5, iv, 0)
            else:
                pltpu.sync_copy(src_v.at[pl.ds(base, _W5)], posv)

                def tv(i, c):
                    valsv[pl.ds(i * 16, 16)] = plsc.load_gather(
                        posv, [lanes * _S5 + i])
                    return c
                lax.fori_loop(0, _S5, tv, 0)
            pltpu.async_copy(score_hbm.at[valsv], keysf, sem).wait()

            def mk(i, c):
                sl = pl.ds(i * 16, 16)
                keysf[sl] = plsc.bitcast(
                    _KEYMAX - plsc.bitcast(keysf[sl], jnp.int32), jnp.float32)
                return c
            lax.fori_loop(0, _S5, mk, 0)

            def zh(i, c):
                histv[pl.ds(i * 16, 16)] = _ZI()
                return c
            lax.fori_loop(0, _RDX, zh, 0)

            def ha(i, c):
                sl = pl.ds(i * 16, 16)
                kv = plsc.bitcast(keysf[sl], jnp.int32)
                d = jnp.bitwise_and(lax.shift_right_logical(kv, shift),
                                    _RDX - 1)
                fl = lanes * _RDX + d
                cnt = plsc.load_gather(histv, [fl])
                plsc.store_scatter(histv, [fl], cnt + 1)
                posv[sl] = cnt
                return c
            lax.fori_loop(0, _S5, ha, 0)

            def wt(j, c):
                acc = _ZI()
                for l in range(16):
                    acc = acc + histv[pl.ds(l * _RDX + j * 16, 16)]
                Tv[pl.ds(j * 16, 16)] = acc
                return c
            lax.fori_loop(0, _RDX // 16, wt, 0)
            pltpu.sync_copy(Tv, Tall_sh.at[idx1k])
            plsc.subcore_barrier()

            wmask = (lanes < sid).astype(jnp.int32)
            carry = jnp.int32(0)
            for g in range(_RDX // 256):
                pltpu.sync_copy(Tall_sh.at[pl.ds(g * 4096, 4096)], tbuf)

                def pb(j, carry):
                    G = _ZI()
                    Wp = _ZI()
                    for m in range(16):
                        wv = tbuf[pl.ds(j * 256 + m * 16, 16)]
                        sel = (lanes == m).astype(jnp.int32)
                        G = G + jnp.sum(wv) * sel
                        Wp = Wp + jnp.sum(wv * wmask) * sel
                    cs = plsc.cumsum(G)
                    Px = carry + cs - G
                    run = Px + Wp
                    for l in range(16):
                        sl_ = pl.ds(l * _RDX + g * 256 + j * 16, 16)
                        tmp = histv[sl_]
                        histv[sl_] = run
                        run = run + tmp
                    return carry + jnp.sum(G)
                carry = lax.fori_loop(0, 16, pb, carry)

            def pc(i, c):
                sl = pl.ds(i * 16, 16)
                kv = plsc.bitcast(keysf[sl], jnp.int32)
                d = jnp.bitwise_and(lax.shift_right_logical(kv, shift),
                                    _RDX - 1)
                base_off = plsc.load_gather(histv, [lanes * _RDX + d])
                posv[sl] = posv[sl] + base_off
                return c
            lax.fori_loop(0, _S5, pc, 0)

            pltpu.sync_copy(valsv, dst_v.at[posv])
            plsc.subcore_barrier()

        base2 = sid * (_R // 16)
        for q in range((_R // 16) // _KCH):
            off2 = base2 + q * _KCH
            pltpu.sync_copy(A_v.at[pl.ds(off2, _KCH)], i2k)
            pltpu.async_copy(score_hbm.at[i2k], s2k, sem).wait()
            pltpu.sync_copy(s2k, rew_hbm.at[pl.ds(off2, _KCH)])
            pltpu.async_copy(ew_hbm.at[i2k], s2k, sem).wait()
            pltpu.sync_copy(s2k, rea_hbm.at[pl.ds(off2, _KCH)])
            pltpu.async_copy(src_hbm.at[i2k], j2k, sem).wait()
            pltpu.sync_copy(j2k, rsrc_hbm.at[pl.ds(off2, _KCH)])
            pltpu.sync_copy(ones2k, pres_r_sh.at[j2k], add=True)
            pltpu.async_copy(dst_hbm.at[i2k], j2k, sem).wait()
            pltpu.sync_copy(j2k, rdst_hbm.at[pl.ds(off2, _KCH)])
            pltpu.sync_copy(ones2k, pres_r_sh.at[j2k], add=True)

        plsc.subcore_barrier()

        @pl.when(sid < 15)
        def _():
            pltpu.sync_copy(pres_r_sh.at[pl.ds(sid * 640, 640)],
                            presr_hbm.at[pl.ds(sid * 640, 640)])

        @pl.when(sid == 15)
        def _():
            pltpu.sync_copy(pres_r_sh.at[pl.ds(9600, 400)],
                            presr_hbm.at[pl.ds(9600, 400)])


def _k7_body(pf_ref, pr_ref, nif_ref, nir_ref):
    r128 = lax.broadcasted_iota(jnp.int32, (_C, _C), 0)
    c128 = lax.broadcasted_iota(jnp.int32, (_C, _C), 1)
    Lm = (r128 <= c128).astype(jnp.float32)
    Jm = jnp.ones((_C, _C), jnp.float32)
    r80 = lax.broadcasted_iota(jnp.int32, (80, 80), 0)
    c80 = lax.broadcasted_iota(jnp.int32, (80, 80), 1)
    SL = (c80 < r80).astype(jnp.float32)

    def ranks(p):
        pfl = (p > 0).astype(jnp.float32)
        incl = (jnp.dot(pfl, Lm, preferred_element_type=jnp.float32)
                + jnp.dot(SL, jnp.dot(pfl, Jm,
                                      preferred_element_type=jnp.float32),
                          preferred_element_type=jnp.float32))
        return jnp.where(pfl > 0, incl - pfl, -1.0).astype(jnp.int32)

    nif_ref[...] = ranks(pf_ref[0] + pf_ref[1])
    nir_ref[...] = ranks(pr_ref[...])


_k7 = pl.pallas_call(
    _k7_body,
    in_specs=[pl.BlockSpec((2, 80, _C), lambda: (0, 0, 0)),
              pl.BlockSpec((80, _C), lambda: (0, 0))],
    out_specs=(pl.BlockSpec((80, _C), lambda: (0, 0)),
               pl.BlockSpec((80, _C), lambda: (0, 0))),
    out_shape=(jax.ShapeDtypeStruct((80, _C), jnp.int32),
               jax.ShapeDtypeStruct((80, _C), jnp.int32)),
)


_NCH = 400
_ECH = _E // 32
_RCH = _R // 32


@functools.partial(
    pl.kernel, mesh=_MESH, compiler_params=_SCP,
    out_type=(jax.ShapeDtypeStruct((_N, _C), jnp.float32),
              jax.ShapeDtypeStruct((_N, _C), jnp.float32),
              jax.ShapeDtypeStruct((_N,), jnp.int32),
              jax.ShapeDtypeStruct((_N,), jnp.int32),
              jax.ShapeDtypeStruct((2, _R), jnp.int32),
              jax.ShapeDtypeStruct((2, _E), jnp.int32)),
    scratch_types=[
        pltpu.VMEM((_NCH,), jnp.int32),
        pltpu.VMEM((_NCH,), jnp.int32),
        pltpu.VMEM((_NCH,), jnp.int32),
        pltpu.VMEM((_NCH, _C), jnp.float32),
        pltpu.VMEM((_ECH,), jnp.int32),
        pltpu.VMEM((_ECH,), jnp.int32),
        pltpu.VMEM((_RCH,), jnp.int32),
        pltpu.VMEM((_RCH,), jnp.int32),
        pltpu.VMEM((_N + 16,), jnp.int32),
        pltpu.VMEM_SHARED((_N + 16,), jnp.int32),
        pltpu.VMEM_SHARED((_N + 16,), jnp.int32),
        pltpu.VMEM_SHARED((_N,), jnp.int32),
        pltpu.VMEM_SHARED((_N,), jnp.int32),
        pltpu.VMEM_SHARED((_N,), jnp.int32),
        pltpu.SemaphoreType.DMA,
    ],
)
def _k8(h_hbm, batch_hbm, nif_hbm, nir_hbm, rsrc_hbm, rdst_hbm, src_hbm,
        dst_hbm, rx_hbm, fx_hbm, rb_hbm, fb_hbm, rei_hbm, fei_hbm,
        nv, posv, valv, rowsv, eb, ob, eb7, ob7, big, sub_f, sub_r, nif_sh,
        nir_sh, batch_sh, sem):
    cid = lax.axis_index("c")
    sid = lax.axis_index("s")
    w = _wid()
    lanes = _LANES()
    @pl.when(sid == 0)
    def _():
        pltpu.sync_copy(nif_hbm, big.at[pl.ds(0, _N)])
        pltpu.sync_copy(big.at[pl.ds(0, _N)], nif_sh)

    @pl.when(sid == 1)
    def _():
        pltpu.sync_copy(nir_hbm, big.at[pl.ds(0, _N)])
        pltpu.sync_copy(big.at[pl.ds(0, _N)], nir_sh)

    @pl.when(sid == 2)
    def _():
        pltpu.sync_copy(batch_hbm, big.at[pl.ds(0, _N)])
        pltpu.sync_copy(big.at[pl.ds(0, _N)], batch_sh)

    @pl.when(sid == 3)
    def _():
        _zero_vmem(big, _N + 16, jnp.int32)
        pltpu.sync_copy(big, sub_f)

    @pl.when(sid == 4)
    def _():
        _zero_vmem(big, _N + 16, jnp.int32)
        pltpu.sync_copy(big, sub_r)

    plsc.subcore_barrier()

    for br in range(2):
        ni_sh = nif_sh if br == 0 else nir_sh
        sub_sh = sub_f if br == 0 else sub_r
        for c in range(_N // _NCH):
            @pl.when(c % 16 == sid)
            def _(c=c, ni_sh=ni_sh, sub_sh=sub_sh):
                pltpu.sync_copy(ni_sh.at[pl.ds(c * _NCH, _NCH)], nv)

                def bld(i, cc):
                    pos = nv[pl.ds(i * 16, 16)]
                    posv[pl.ds(i * 16, 16)] = jnp.where(
                        pos < 0, _N + lanes, pos)
                    valv[pl.ds(i * 16, 16)] = c * _NCH + i * 16 + lanes
                    return cc
                lax.fori_loop(0, _NCH // 16, bld, 0)
                pltpu.sync_copy(valv, sub_sh.at[posv])
    plsc.subcore_barrier()

    for t in range(2):
        if t == 0:
            cidx = w
        else:
            cidx = 32 + sid + 9 * cid

        @pl.when(jnp.logical_and(t == 0 or sid < 9, cidx < _N // _NCH))
        def _(cidx=cidx):
            off = cidx * _NCH
            pltpu.sync_copy(sub_r.at[pl.ds(off, _NCH)], nv)
            pltpu.async_copy(h_hbm.at[nv], rowsv, sem).wait()
            pltpu.sync_copy(rowsv, rx_hbm.at[pl.ds(off, _NCH)])
            pltpu.sync_copy(batch_sh.at[nv], posv)
            pltpu.sync_copy(posv, rb_hbm.at[pl.ds(off, _NCH)])

        @pl.when(jnp.logical_and(t == 0 or sid < 9,
                                 jnp.logical_and(cidx >= _N // _NCH,
                                                 cidx < 2 * (_N // _NCH))))
        def _(cidx=cidx):
            off = (cidx - _N // _NCH) * _NCH
            pltpu.sync_copy(sub_f.at[pl.ds(off, _NCH)], nv)
            pltpu.async_copy(h_hbm.at[nv], rowsv, sem).wait()
            pltpu.sync_copy(rowsv, fx_hbm.at[pl.ds(off, _NCH)])
            pltpu.sync_copy(batch_sh.at[nv], posv)
            pltpu.sync_copy(posv, fb_hbm.at[pl.ds(off, _NCH)])

    eoff = w * _ECH
    pltpu.sync_copy(src_hbm.at[pl.ds(eoff, _ECH)], eb)
    pltpu.sync_copy(nif_sh.at[eb], ob)
    pltpu.sync_copy(ob, fei_hbm.at[0, pl.ds(eoff, _ECH)])
    pltpu.sync_copy(dst_hbm.at[pl.ds(eoff, _ECH)], eb)
    pltpu.sync_copy(nif_sh.at[eb], ob)
    pltpu.sync_copy(ob, fei_hbm.at[1, pl.ds(eoff, _ECH)])
    roff = w * _RCH
    pltpu.sync_copy(rsrc_hbm.at[pl.ds(roff, _RCH)], eb7)
    pltpu.sync_copy(nir_sh.at[eb7], ob7)
    pltpu.sync_copy(ob7, rei_hbm.at[0, pl.ds(roff, _RCH)])
    pltpu.sync_copy(rdst_hbm.at[pl.ds(roff, _RCH)], eb7)
    pltpu.sync_copy(nir_sh.at[eb7], ob7)
    pltpu.sync_copy(ob7, rei_hbm.at[1, pl.ds(roff, _RCH)])


def kernel(x, edge_index, edge_attr, batch, edge_score, c1_w1, c1_b1, c1_w2,
           c1_w3, c1_b3, c2_w1, c2_b1, c2_w2, c2_w3, c2_b3):
    src = edge_index[0]
    dst = edge_index[1]
    ew = edge_attr.reshape(-1)
    xe = jnp.concatenate(
        [x, jnp.ones((_N, 1), jnp.float32), jnp.zeros((_N, 11), jnp.float32)],
        axis=1)

    acc1, pres_f = _k1(xe, edge_index, ew)

    zw = jnp.zeros((16, _C), jnp.float32)
    w1a = zw.at[0:4].set(c1_w1).at[4].set(c1_b1)
    w2a = zw.at[0:4].set(c1_w2)
    w3a = zw.at[0:4].set(c1_w3).at[4].set(c1_b3)
    h1 = _k2(acc1, xe, w1a, w2a, w3a)

    acc2 = _k3(h1, edge_index, ew)
    h = _k4(acc2, acc1, h1, c2_w1, c2_w2, c2_w3, c2_b1[None, :],
            c2_b3[None, :])

    r_ew, r_ea, r_src, r_dst, pres_r = _k5(edge_score, ew, src, dst)

    pf = jnp.pad(pres_f, ((0, 0), (0, 10240 - _N))).reshape(2, 80, _C)
    pr = jnp.pad(pres_r, (0, 10240 - _N)).reshape(80, _C)
    nif80, nir80 = _k7(pf, pr)
    nif = nif80.reshape(-1)[:_N]
    nir = nir80.reshape(-1)[:_N]

    r_x, f_x, r_batch, f_batch, r_ei2, f_ei2 = _k8(
        h, batch, nif, nir, r_src, r_dst, src, dst)

    return ((r_x, r_ei2, r_ea, r_ew, r_batch),
            (f_x, f_ei2, ew, edge_score, f_batch),
            edge_score)
